# Initial kernel scaffold; baseline (speedup 1.0000x reference)
import jax
import jax.numpy as jnp
from jax.experimental import pallas as pl


def kernel(x_obj, edge_index_obj, edge_weight_obj, x_feas, edge_index_feas, edge_weight_feas, batch_var, binary_idx, eps, params):
    raise NotImplementedError("write your pallas kernel here")


# speedup vs baseline: 7.9655x; 7.9655x over previous
import functools

import jax
import jax.numpy as jnp
from jax import lax
from jax.experimental import pallas as pl
from jax.experimental.pallas import tpu as pltpu
from jax.experimental.pallas import tpu_sc as plsc

NC = 2
NS = 16
CH = 128
RB = 800
NG = 16
HF = 16


def _align8(v):
    return -(-v // 8) * 8


def _mesh():
    return plsc.VectorSubcoreMesh(core_axis_name="c", subcore_axis_name="s")


def _sc_params():
    return pltpu.CompilerParams(use_tc_tiling_on_sc=False)


@functools.cache
def _make_deg(E, N):
    ept = E // (NC * NS)
    assert ept * NC * NS == E and ept % 16 == 0
    nfull, tail = divmod(ept, CH)
    assert tail % 16 == 0
    rpt = _align8(-(-N // NS))
    npad = rpt * NS
    nzr, ztr = divmod(rpt, CH)

    scratch = [
        pltpu.VMEM((CH,), jnp.int32),
        pltpu.VMEM((CH,), jnp.float32),
        pltpu.VMEM((CH, HF), jnp.float32),
        pltpu.VMEM((max(tail, 16),), jnp.int32),
        pltpu.VMEM((max(tail, 16),), jnp.float32),
        pltpu.VMEM((max(tail, 16), HF), jnp.float32),
        pltpu.VMEM((CH, HF), jnp.float32),
        pltpu.VMEM_SHARED((npad, HF), jnp.float32),
        pltpu.SemaphoreType.DMA,
    ]

    @functools.partial(
        pl.kernel,
        out_type=[jax.ShapeDtypeStruct((npad, HF), jnp.float32)] * NC,
        mesh=_mesh(),
        scratch_types=scratch,
        compiler_params=_sc_params(),
    )
    def deg_kernel(dst_hbm, ew_hbm, out0, out1, dst_v, ew_v, rows_v,
                   dst_t, ew_t, rows_t, zb, acc, sem):
        c = lax.axis_index("c")
        s = lax.axis_index("s")
        wid = s * NC + c

        def zloop(i, _):
            zb[i, :] = jnp.zeros((HF,), jnp.float32)
            return 0
        lax.fori_loop(0, CH, zloop, 0)

        row0 = pl.multiple_of(s * rpt, 8)

        def zcp(i, _):
            r = pl.multiple_of(row0 + i * CH, 8)
            pltpu.sync_copy(zb, acc.at[pl.ds(r, CH), :])
            return 0
        lax.fori_loop(0, nzr, zcp, 0)
        if ztr:
            r = pl.multiple_of(row0 + nzr * CH, 8)
            pltpu.sync_copy(zb.at[pl.ds(0, ztr), :], acc.at[pl.ds(r, ztr), :])
        plsc.subcore_barrier()

        def oloop(i, _):
            zb[i, :] = jnp.ones((HF,), jnp.float32)
            return 0
        lax.fori_loop(0, CH, oloop, 0)

        base = wid * ept

        def chunk(o, n, di, wi, rv):
            o = pl.multiple_of(o, 8)
            pltpu.sync_copy(dst_hbm.at[pl.ds(o, n)], di)
            pltpu.sync_copy(ew_hbm.at[pl.ds(o, n)], wi)

            def fill(j, _):
                wv = wi[pl.ds(j * 16, 16)]
                for k in range(16):
                    w16 = wv.at[jnp.full((16,), k, jnp.int32)].get(
                        mode="promise_in_bounds")
                    r = j * 16 + k
                    rv[r, :] = zb[r, :] * w16
                return 0
            lax.fori_loop(0, n // 16, fill, 0)
            pltpu.sync_copy(rv, acc.at[di], add=True)

        def body(i, _):
            chunk(base + i * CH, CH, dst_v, ew_v, rows_v)
            return 0
        lax.fori_loop(0, nfull, body, 0)
        if tail:
            chunk(base + nfull * CH, tail, dst_t, ew_t, rows_t)
        plsc.subcore_barrier()

        def flush(out_ref):
            def fcp(i, _):
                r = pl.multiple_of(row0 + i * CH, 8)
                pltpu.sync_copy(acc.at[pl.ds(r, CH), :], zb)
                pltpu.sync_copy(zb, out_ref.at[pl.ds(r, CH), :])
                return 0
            lax.fori_loop(0, nzr, fcp, 0)
            if ztr:
                r = pl.multiple_of(row0 + nzr * CH, 8)
                pltpu.sync_copy(acc.at[pl.ds(r, ztr), :], zb.at[pl.ds(0, ztr), :])
                pltpu.sync_copy(zb.at[pl.ds(0, ztr), :],
                                out_ref.at[pl.ds(r, ztr), :])

        @pl.when(c == 0)
        def _():
            flush(out0)

        @pl.when(c == 1)
        def _():
            flush(out1)

    return deg_kernel


@functools.cache
def _make_segsum(E, N):
    ept = E // NS
    assert ept * NS == E and ept % 8 == 0
    nfull, tail = divmod(ept, CH)
    rpt = _align8(-(-N // NS))
    npad = rpt * NS
    nzr, ztr = divmod(rpt, CH)

    scratch = [
        pltpu.VMEM((CH,), jnp.int32),
        pltpu.VMEM((CH,), jnp.int32),
        pltpu.VMEM((CH,), jnp.float32),
        pltpu.VMEM((CH, HF), jnp.float32),
        pltpu.VMEM((max(tail, 8),), jnp.int32),
        pltpu.VMEM((max(tail, 8),), jnp.int32),
        pltpu.VMEM((max(tail, 8),), jnp.float32),
        pltpu.VMEM((max(tail, 8), HF), jnp.float32),
        pltpu.VMEM((CH, HF), jnp.float32),
        pltpu.VMEM_SHARED((npad, HF), jnp.float32),
        pltpu.SemaphoreType.DMA,
    ]

    @functools.partial(
        pl.kernel,
        out_type=[jax.ShapeDtypeStruct((npad, HF), jnp.float32)] * NC,
        mesh=_mesh(),
        scratch_types=scratch,
        compiler_params=_sc_params(),
    )
    def seg_kernel(g0, g1, src_hbm, dst_hbm, ew_hbm, o0, o1,
                   src_v, dst_v, ew_v, rows_v, src_t, dst_t, ew_t, rows_t,
                   zb, acc, sem):
        c = lax.axis_index("c")
        s = lax.axis_index("s")

        def zloop(i, _):
            zb[i, :] = jnp.zeros((HF,), jnp.float32)
            return 0
        lax.fori_loop(0, CH, zloop, 0)

        row0 = pl.multiple_of(s * rpt, 8)

        def zcp(i, _):
            r = pl.multiple_of(row0 + i * CH, 8)
            pltpu.sync_copy(zb, acc.at[pl.ds(r, CH), :])
            return 0
        lax.fori_loop(0, nzr, zcp, 0)
        if ztr:
            r = pl.multiple_of(row0 + nzr * CH, 8)
            pltpu.sync_copy(zb.at[pl.ds(0, ztr), :], acc.at[pl.ds(r, ztr), :])
        plsc.subcore_barrier()

        base = s * ept

        def run(g_ref):
            def chunk(o, n, si, di, wi, rv):
                o = pl.multiple_of(o, 8)
                pltpu.sync_copy(src_hbm.at[pl.ds(o, n)], si)
                pltpu.sync_copy(dst_hbm.at[pl.ds(o, n)], di)
                pltpu.sync_copy(ew_hbm.at[pl.ds(o, n)], wi)
                pltpu.async_copy(g_ref.at[si], rv, sem).wait()
                assert n % 16 == 0

                def scale(j, _):
                    wv = wi[pl.ds(j * 16, 16)]
                    for k in range(16):
                        w16 = wv.at[jnp.full((16,), k, jnp.int32)].get(
                            mode="promise_in_bounds")
                        r = j * 16 + k
                        rv[r, :] = rv[r, :] * w16
                    return 0
                lax.fori_loop(0, n // 16, scale, 0)
                pltpu.sync_copy(rv, acc.at[di], add=True)

            def body(i, _):
                chunk(base + i * CH, CH, src_v, dst_v, ew_v, rows_v)
                return 0
            lax.fori_loop(0, nfull, body, 0)
            if tail:
                chunk(base + nfull * CH, tail, src_t, dst_t, ew_t, rows_t)

        @pl.when(c == 0)
        def _():
            run(g0)

        @pl.when(c == 1)
        def _():
            run(g1)
        plsc.subcore_barrier()

        def flush(out_ref):
            def fcp(i, _):
                r = pl.multiple_of(row0 + i * CH, 8)
                pltpu.sync_copy(acc.at[pl.ds(r, CH), :], zb)
                pltpu.sync_copy(zb, out_ref.at[pl.ds(r, CH), :])
                return 0
            lax.fori_loop(0, nzr, fcp, 0)
            if ztr:
                r = pl.multiple_of(row0 + nzr * CH, 8)
                pltpu.sync_copy(acc.at[pl.ds(r, ztr), :], zb.at[pl.ds(0, ztr), :])
                pltpu.sync_copy(zb.at[pl.ds(0, ztr), :],
                                out_ref.at[pl.ds(r, ztr), :])

        @pl.when(c == 0)
        def _():
            flush(o0)

        @pl.when(c == 1)
        def _():
            flush(o1)

    return seg_kernel


@functools.cache
def _make_take(NI):
    nck, tailc = divmod(NI, CH)
    rounds = -(-nck // (NC * NS))

    scratch = [
        pltpu.VMEM((CH,), jnp.int32),
        pltpu.VMEM((CH,), jnp.float32),
        pltpu.VMEM((max(tailc, 8),), jnp.int32),
        pltpu.VMEM((max(tailc, 8),), jnp.float32),
        pltpu.SemaphoreType.DMA,
    ]

    @functools.partial(
        pl.kernel,
        out_type=jax.ShapeDtypeStruct((NI,), jnp.float32),
        mesh=_mesh(),
        scratch_types=scratch,
        compiler_params=_sc_params(),
    )
    def take_kernel(tab, idx, out, idx_v, val_v, idx_t, val_t, sem):
        c = lax.axis_index("c")
        s = lax.axis_index("s")
        wid = s * NC + c
        for k in range(rounds):
            cid = wid + k * NC * NS

            @pl.when(cid < nck)
            def _(cid=cid):
                o = pl.multiple_of(cid * CH, 8)
                pltpu.sync_copy(idx.at[pl.ds(o, CH)], idx_v)
                pltpu.async_copy(tab.at[idx_v], val_v, sem).wait()
                pltpu.sync_copy(val_v, out.at[pl.ds(o, CH)])
        if tailc:
            @pl.when(wid == 0)
            def _():
                o = pl.multiple_of(nck * CH, 8)
                pltpu.sync_copy(idx.at[pl.ds(o, tailc)], idx_t)
                pltpu.async_copy(tab.at[idx_t], val_t, sem).wait()
                pltpu.sync_copy(val_t, out.at[pl.ds(o, tailc)])

    return take_kernel


@functools.cache
def _make_prep(N, DI, H):
    nb = N // RB

    @functools.partial(
        pl.pallas_call,
        grid=(nb,),
        in_specs=[
            pl.BlockSpec((RB, DI), lambda i: (i, 0)),
            pl.BlockSpec((DI, H), lambda i: (0, 0)),
            pl.BlockSpec((RB, 1), lambda i: (i, 0)),
            pl.BlockSpec((RB, 1), lambda i: (i, 0)),
        ],
        out_specs=[
            pl.BlockSpec((RB, 1), lambda i: (i, 0)),
            pl.BlockSpec((RB, HF), lambda i: (i, 0)),
            pl.BlockSpec((RB, HF), lambda i: (i, 0)),
        ],
        out_shape=[
            jax.ShapeDtypeStruct((N, 1), jnp.float32),
            jax.ShapeDtypeStruct((N, HF), jnp.float32),
            jax.ShapeDtypeStruct((N, HF), jnp.float32),
        ],
    )
    def prep(x_ref, w_ref, d0, d1, dinv_o, g0_o, g1_o):
        deg = d0[...] + d1[...] + 1.0
        dinv = lax.rsqrt(deg)
        g = jnp.dot(x_ref[...], w_ref[...],
                    preferred_element_type=jnp.float32) * dinv
        dinv_o[...] = dinv
        g0_o[...] = g[:, :HF]
        g1_o[...] = g[:, HF:]

    return prep


@functools.cache
def _make_mid(N, H):
    nb = N // RB
    half = pl.BlockSpec((RB, HF), lambda i: (i, 0))

    @functools.partial(
        pl.pallas_call,
        grid=(nb,),
        in_specs=[
            half, half, half, half,
            pl.BlockSpec((RB, 1), lambda i: (i, 0)),
            pl.BlockSpec((1, H), lambda i: (0, 0)),
            pl.BlockSpec((H, H), lambda i: (0, 0)),
        ],
        out_specs=[half, half],
        out_shape=[
            jax.ShapeDtypeStruct((N, HF), jnp.float32),
            jax.ShapeDtypeStruct((N, HF), jnp.float32),
        ],
    )
    def mid(s0, s1, g0, g1, dinv, b, w, o0, o1):
        t = jnp.concatenate([s0[...] + g0[...], s1[...] + g1[...]], axis=1)
        h = jnp.maximum(t * dinv[...] + b[...], 0.0)
        g2 = jnp.dot(h, w[...], preferred_element_type=jnp.float32) * dinv[...]
        o0[...] = g2[:, :HF]
        o1[...] = g2[:, HF:]

    return mid


@functools.cache
def _make_fin(N, H):
    nb = N // RB
    half = pl.BlockSpec((RB, HF), lambda i: (i, 0))

    @functools.partial(
        pl.pallas_call,
        grid=(nb,),
        in_specs=[
            half, half, half, half,
            pl.BlockSpec((RB, 1), lambda i: (i, 0)),
            pl.BlockSpec((1, H), lambda i: (0, 0)),
        ],
        out_specs=pl.BlockSpec((RB, H), lambda i: (i, 0)),
        out_shape=jax.ShapeDtypeStruct((N, H), jnp.float32),
    )
    def fin(s0, s1, g0, g1, dinv, b, z_o):
        t = jnp.concatenate([s0[...] + g0[...], s1[...] + g1[...]], axis=1)
        z_o[...] = jnp.maximum(t * dinv[...] + b[...], 0.0)

    return fin


@functools.cache
def _make_head(NV, NTOT, H, CC, DEC):
    vb = NV // RB
    nb = NTOT // RB
    NK = NTOT - NV

    def cw(shape):
        return pl.BlockSpec(shape, lambda i: (0,) * len(shape))

    @functools.partial(
        pl.pallas_call,
        grid=(nb,),
        in_specs=[
            pl.BlockSpec((RB, H), lambda i: (jnp.minimum(i, vb - 1), 0)),
            pl.BlockSpec((RB, H), lambda i: (i, 0)),
            pl.BlockSpec((RB, CC), lambda i: (i, 0)),
            pl.BlockSpec((1, 1, RB), lambda i: (jnp.minimum(i, vb - 1), 0, 0)),
            cw((CC, CC)), cw((1, CC)), cw((CC, CC)), cw((1, CC)),
            cw((CC, DEC)), cw((1, DEC)), cw((1, DEC)), cw((1, 1)),
            cw((CC, DEC)), cw((1, DEC)), cw((1, DEC)), cw((1, 1)),
            cw((CC, DEC)), cw((1, DEC)), cw((1, DEC)), cw((1, 1)),
        ],
        out_specs=[
            pl.BlockSpec((RB, CC), lambda i: (i, 0)),
            pl.BlockSpec((RB, CC), lambda i: (i, 0)),
            pl.BlockSpec((RB, 1), lambda i: (jnp.minimum(i, vb - 1), 0)),
            pl.BlockSpec((RB, 1), lambda i: (jnp.maximum(i - vb, 0), 0)),
            pl.BlockSpec((RB, 1), lambda i: (jnp.minimum(i, vb - 1), 0)),
            pl.BlockSpec((NG, CC), lambda i: (0, 0)),
            pl.BlockSpec((NG, CC), lambda i: (0, 0)),
        ],
        out_shape=[
            jax.ShapeDtypeStruct((NTOT, CC), jnp.float32),
            jax.ShapeDtypeStruct((NTOT, CC), jnp.float32),
            jax.ShapeDtypeStruct((NV, 1), jnp.float32),
            jax.ShapeDtypeStruct((NK, 1), jnp.float32),
            jax.ShapeDtypeStruct((NV, 1), jnp.float32),
            jax.ShapeDtypeStruct((NG, CC), jnp.float32),
            jax.ShapeDtypeStruct((NG, CC), jnp.float32),
        ],
    )
    def head(zo, zf, eps, bv, muw, mub, lvw, lvb,
             dxw1, dxb1, dxw2, dxb2, dkw1, dkb1, dkw2, dkb2,
             diw1, dib1, diw2, dib2,
             zmu_o, zlv_o, xh_o, pk_o, ig_o, ps_o, pn_o):
        i = pl.program_id(0)
        isv = i < vb
        f = zf[...]
        c0 = jnp.where(isv, zo[...], f)
        c1 = jnp.where(isv, f, jnp.zeros_like(f))
        z = jnp.concatenate([c0, c1], axis=1)
        mu = jnp.dot(z, muw[...], preferred_element_type=jnp.float32) + mub[...]
        lv = jnp.dot(z, lvw[...], preferred_element_type=jnp.float32) + lvb[...]
        zmu_o[...] = mu
        zlv_o[...] = lv
        zz = mu + jnp.exp(0.5 * lv) * eps[...]

        @pl.when(i == 0)
        def _():
            ps_o[...] = jnp.zeros((NG, CC), jnp.float32)
            pn_o[...] = jnp.zeros((NG, CC), jnp.float32)

        @pl.when(isv)
        def _():
            h = jnp.maximum(
                jnp.dot(zz, dxw1[...], preferred_element_type=jnp.float32)
                + dxb1[...], 0.0)
            xh_o[...] = jnp.sum(h * dxw2[...], axis=1, keepdims=True) + dxb2[0, 0]
            hi = jnp.maximum(
                jnp.dot(zz, diw1[...], preferred_element_type=jnp.float32)
                + dib1[...], 0.0)
            logit = jnp.sum(hi * diw2[...], axis=1, keepdims=True) + dib2[0, 0]
            ig_o[...] = 1.0 / (1.0 + jnp.exp(-logit))
            grp = bv[0, 0, :]
            m = (lax.broadcasted_iota(jnp.int32, (NG, RB), 0)
                 == grp[None, :]).astype(jnp.float32)
            ps_o[...] += jnp.dot(m, zz, preferred_element_type=jnp.float32)
            pn_o[...] += jnp.dot(m, jnp.ones((RB, CC), jnp.float32),
                                 preferred_element_type=jnp.float32)

        @pl.when(jnp.logical_not(isv))
        def _():
            hk = jnp.maximum(
                jnp.dot(zz, dkw1[...], preferred_element_type=jnp.float32)
                + dkb1[...], 0.0)
            pk_o[...] = jnp.sum(hk * dkw2[...], axis=1, keepdims=True) + dkb2[0, 0]

    return head


@functools.cache
def _make_cost(CC, DEC):
    @functools.partial(
        pl.pallas_call,
        grid=(1,),
        in_specs=[
            pl.BlockSpec((NG, CC), lambda i: (0, 0)),
            pl.BlockSpec((NG, CC), lambda i: (0, 0)),
            pl.BlockSpec((CC, DEC), lambda i: (0, 0)),
            pl.BlockSpec((1, DEC), lambda i: (0, 0)),
            pl.BlockSpec((1, DEC), lambda i: (0, 0)),
            pl.BlockSpec((1, 1), lambda i: (0, 0)),
        ],
        out_specs=pl.BlockSpec((NG, 1), lambda i: (0, 0)),
        out_shape=jax.ShapeDtypeStruct((NG, 1), jnp.float32),
    )
    def cost(ps, pn, w1, b1, w2, b2, out):
        pooled = ps[...] / jnp.maximum(pn[...], 1.0)
        h = jnp.maximum(
            jnp.dot(pooled, w1[...], preferred_element_type=jnp.float32)
            + b1[...], 0.0)
        out[...] = jnp.sum(h * w2[...], axis=1, keepdims=True) + b2[0, 0]

    return cost


def _encode(x, src, dst, ew, W1, b1, W2, b2):
    n = x.shape[0]
    e = ew.shape[0]
    h = W1.shape[1]
    d0, d1 = _make_deg(e, n)(dst, ew)
    dinv, g0, g1 = _make_prep(n, x.shape[1], h)(
        x, W1, d0[:n, :1], d1[:n, :1])
    s0, s1 = _make_segsum(e, n)(g0, g1, src, dst, ew)
    h0, h1 = _make_mid(n, h)(
        s0[:n], s1[:n], g0, g1, dinv, b1.reshape(1, -1), W2)
    t0, t1 = _make_segsum(e, n)(h0, h1, src, dst, ew)
    return _make_fin(n, h)(t0[:n], t1[:n], h0, h1, dinv, b2.reshape(1, -1))


def kernel(x_obj, edge_index_obj, edge_weight_obj, x_feas, edge_index_feas,
           edge_weight_feas, batch_var, binary_idx, eps, params):
    p = params
    n_var = batch_var.shape[0]
    n_tot = x_feas.shape[0]
    cc = eps.shape[1]
    dec = p["dx_W1"].shape[1]

    z_obj = _encode(x_obj, edge_index_obj[0], edge_index_obj[1],
                    edge_weight_obj, p["obj_W1"], p["obj_b1"],
                    p["obj_W2"], p["obj_b2"])
    zf = _encode(x_feas, edge_index_feas[0], edge_index_feas[1],
                 edge_weight_feas, p["cons_W1"], p["cons_b1"],
                 p["cons_W2"], p["cons_b2"])

    bv3 = batch_var.reshape(n_var // RB, 1, RB)
    zmu, zlv, xh, pk, ig, ps, pn = _make_head(
        n_var, n_tot, z_obj.shape[1], cc, dec)(
        z_obj, zf, eps, bv3,
        p["mu_W"], p["mu_b"].reshape(1, -1),
        p["lv_W"], p["lv_b"].reshape(1, -1),
        p["dx_W1"], p["dx_b1"].reshape(1, -1),
        p["dx_W2"].reshape(1, -1), p["dx_b2"].reshape(1, 1),
        p["dk_W1"], p["dk_b1"].reshape(1, -1),
        p["dk_W2"].reshape(1, -1), p["dk_b2"].reshape(1, 1),
        p["di_W1"], p["di_b1"].reshape(1, -1),
        p["di_W2"].reshape(1, -1), p["di_b2"].reshape(1, 1))

    cost = _make_cost(cc, dec)(
        ps, pn, p["dc_W1"], p["dc_b1"].reshape(1, -1),
        p["dc_W2"].reshape(1, -1), p["dc_b2"].reshape(1, 1))
    pint = _make_take(binary_idx.shape[0])(ig.reshape(-1), binary_idx)
    return (xh.reshape(-1), cost.reshape(-1), pk.reshape(-1), pint, zmu, zlv)


# speedup vs baseline: 18.1246x; 2.2754x over previous
import functools

import jax
import jax.numpy as jnp
from jax import lax
from jax.experimental import pallas as pl
from jax.experimental.pallas import tpu as pltpu
from jax.experimental.pallas import tpu_sc as plsc

NC = 2
NS = 16
CH = 128
RB = 800
NG = 16
HF = 16


def _align8(v):
    return -(-v // 8) * 8


def _mesh():
    return plsc.VectorSubcoreMesh(core_axis_name="c", subcore_axis_name="s")


def _sc_params():
    return pltpu.CompilerParams(use_tc_tiling_on_sc=False)


@functools.cache
def _make_deg(E, N):
    ept = E // (NC * NS)
    assert ept * NC * NS == E and ept % 16 == 0
    nfull, tail = divmod(ept, CH)
    assert tail % 16 == 0
    rpt = _align8(-(-N // NS))
    npad = rpt * NS
    nzr, ztr = divmod(rpt, CH)

    scratch = [
        pltpu.VMEM((CH,), jnp.int32),
        pltpu.VMEM((CH,), jnp.float32),
        pltpu.VMEM((CH, HF), jnp.float32),
        pltpu.VMEM((CH,), jnp.int32),
        pltpu.VMEM((CH,), jnp.float32),
        pltpu.VMEM((CH, HF), jnp.float32),
        pltpu.VMEM((max(tail, 16),), jnp.int32),
        pltpu.VMEM((max(tail, 16),), jnp.float32),
        pltpu.VMEM((max(tail, 16), HF), jnp.float32),
        pltpu.VMEM((CH, HF), jnp.float32),
        pltpu.VMEM_SHARED((npad, HF), jnp.float32),
        pltpu.SemaphoreType.DMA,
        pltpu.SemaphoreType.DMA,
        pltpu.SemaphoreType.DMA,
    ]

    @functools.partial(
        pl.kernel,
        out_type=[jax.ShapeDtypeStruct((npad, HF), jnp.float32)] * NC,
        mesh=_mesh(),
        scratch_types=scratch,
        compiler_params=_sc_params(),
    )
    def deg_kernel(dst_hbm, ew_hbm, out0, out1, di0, wi0, rv0, di1, wi1, rv1,
                   dst_t, ew_t, rows_t, zb, acc, ls0, ls1, sem):
        c = lax.axis_index("c")
        s = lax.axis_index("s")
        wid = s * NC + c
        dib, wib, rvb, lsb = [di0, di1], [wi0, wi1], [rv0, rv1], [ls0, ls1]

        def zloop(i, _):
            zb[i, :] = jnp.zeros((HF,), jnp.float32)
            return 0
        lax.fori_loop(0, CH, zloop, 0)

        row0 = pl.multiple_of(s * rpt, 8)

        def zcp(i, _):
            r = pl.multiple_of(row0 + i * CH, 8)
            pltpu.sync_copy(zb, acc.at[pl.ds(r, CH), :])
            return 0
        lax.fori_loop(0, nzr, zcp, 0)
        if ztr:
            r = pl.multiple_of(row0 + nzr * CH, 8)
            pltpu.sync_copy(zb.at[pl.ds(0, ztr), :], acc.at[pl.ds(r, ztr), :])
        plsc.subcore_barrier()

        def oloop(i, _):
            zb[i, :] = jnp.ones((HF,), jnp.float32)
            return 0
        lax.fori_loop(0, CH, oloop, 0)

        base = wid * ept

        def issue_loads(i, b):
            o = pl.multiple_of(base + i * CH, 8)
            pltpu.async_copy(dst_hbm.at[pl.ds(o, CH)], dib[b], lsb[b])
            pltpu.async_copy(ew_hbm.at[pl.ds(o, CH)], wib[b], lsb[b])

        def wait_loads(b):
            o0_ = pl.multiple_of(base, 8)
            pltpu.make_async_copy(
                dst_hbm.at[pl.ds(o0_, CH)], dib[b], lsb[b]).wait()
            pltpu.make_async_copy(
                ew_hbm.at[pl.ds(o0_, CH)], wib[b], lsb[b]).wait()

        def fill_scatter(b, n, di, wi, rv):
            def fill(j, _):
                wv = wi[pl.ds(j * 16, 16)]
                for k in range(16):
                    w16 = wv.at[jnp.full((16,), k, jnp.int32)].get(
                        mode="promise_in_bounds")
                    r = j * 16 + k
                    rv[r, :] = zb[r, :] * w16
                return 0
            lax.fori_loop(0, n // 16, fill, 0)
            pltpu.sync_copy(rv, acc.at[di], add=True)

        def step(i, b):
            wait_loads(b)
            fill_scatter(b, CH, dib[b], wib[b], rvb[b])

            @pl.when(i + 2 < nfull)
            def _():
                issue_loads(i + 2, b)

        issue_loads(0, 0)
        if nfull > 1:
            issue_loads(1, 1)

        def pair(p, _):
            step(2 * p, 0)
            step(2 * p + 1, 1)
            return 0
        lax.fori_loop(0, nfull // 2, pair, 0)
        if nfull % 2:
            step(nfull - 1, (nfull - 1) % 2)

        if tail:
            o = pl.multiple_of(base + nfull * CH, 8)
            pltpu.sync_copy(dst_hbm.at[pl.ds(o, tail)], dst_t)
            pltpu.sync_copy(ew_hbm.at[pl.ds(o, tail)], ew_t)
            fill_scatter(0, tail, dst_t, ew_t, rows_t)
        plsc.subcore_barrier()

        def flush(out_ref):
            def fcp(i, _):
                r = pl.multiple_of(row0 + i * CH, 8)
                pltpu.sync_copy(acc.at[pl.ds(r, CH), :], zb)
                pltpu.sync_copy(zb, out_ref.at[pl.ds(r, CH), :])
                return 0
            lax.fori_loop(0, nzr, fcp, 0)
            if ztr:
                r = pl.multiple_of(row0 + nzr * CH, 8)
                pltpu.sync_copy(acc.at[pl.ds(r, ztr), :], zb.at[pl.ds(0, ztr), :])
                pltpu.sync_copy(zb.at[pl.ds(0, ztr), :],
                                out_ref.at[pl.ds(r, ztr), :])

        @pl.when(c == 0)
        def _():
            flush(out0)

        @pl.when(c == 1)
        def _():
            flush(out1)

    return deg_kernel


@functools.cache
def _make_segsum(E, N):
    ept = E // NS
    assert ept * NS == E and ept % 8 == 0
    nfull, tail = divmod(ept, CH)
    rpt = _align8(-(-N // NS))
    npad = rpt * NS
    nzr, ztr = divmod(rpt, CH)

    scratch = [
        pltpu.VMEM((CH,), jnp.int32),
        pltpu.VMEM((CH,), jnp.int32),
        pltpu.VMEM((CH,), jnp.float32),
        pltpu.VMEM((CH, HF), jnp.float32),
        pltpu.VMEM((CH,), jnp.int32),
        pltpu.VMEM((CH,), jnp.int32),
        pltpu.VMEM((CH,), jnp.float32),
        pltpu.VMEM((CH, HF), jnp.float32),
        pltpu.VMEM((max(tail, 16),), jnp.int32),
        pltpu.VMEM((max(tail, 16),), jnp.int32),
        pltpu.VMEM((max(tail, 16),), jnp.float32),
        pltpu.VMEM((max(tail, 16), HF), jnp.float32),
        pltpu.VMEM((CH, HF), jnp.float32),
        pltpu.VMEM_SHARED((npad, HF), jnp.float32),
        pltpu.SemaphoreType.DMA,
        pltpu.SemaphoreType.DMA,
        pltpu.SemaphoreType.DMA,
        pltpu.SemaphoreType.DMA,
        pltpu.SemaphoreType.DMA,
    ]

    @functools.partial(
        pl.kernel,
        out_type=[jax.ShapeDtypeStruct((npad, HF), jnp.float32)] * NC,
        mesh=_mesh(),
        scratch_types=scratch,
        compiler_params=_sc_params(),
    )
    def seg_kernel(g0, g1, src_hbm, dst_hbm, ew_hbm, o0, o1,
                   si0, di0, wi0, rv0, si1, di1, wi1, rv1,
                   src_t, dst_t, ew_t, rows_t,
                   zb, acc, ls0, ls1, gs0, gs1, sem):
        c = lax.axis_index("c")
        s = lax.axis_index("s")
        sib, dib, wib, rvb = [si0, si1], [di0, di1], [wi0, wi1], [rv0, rv1]
        lsb, gsb = [ls0, ls1], [gs0, gs1]

        def zloop(i, _):
            zb[i, :] = jnp.zeros((HF,), jnp.float32)
            return 0
        lax.fori_loop(0, CH, zloop, 0)

        row0 = pl.multiple_of(s * rpt, 8)

        def zcp(i, _):
            r = pl.multiple_of(row0 + i * CH, 8)
            pltpu.sync_copy(zb, acc.at[pl.ds(r, CH), :])
            return 0
        lax.fori_loop(0, nzr, zcp, 0)
        if ztr:
            r = pl.multiple_of(row0 + nzr * CH, 8)
            pltpu.sync_copy(zb.at[pl.ds(0, ztr), :], acc.at[pl.ds(r, ztr), :])
        plsc.subcore_barrier()

        base = s * ept

        def run(g_ref):
            def issue_loads(i, b):
                o = pl.multiple_of(base + i * CH, 8)
                pltpu.async_copy(src_hbm.at[pl.ds(o, CH)], sib[b], lsb[b])
                pltpu.async_copy(dst_hbm.at[pl.ds(o, CH)], dib[b], lsb[b])
                pltpu.async_copy(ew_hbm.at[pl.ds(o, CH)], wib[b], lsb[b])

            def wait_loads(b):
                o0_ = pl.multiple_of(base, 8)
                pltpu.make_async_copy(
                    src_hbm.at[pl.ds(o0_, CH)], sib[b], lsb[b]).wait()
                pltpu.make_async_copy(
                    dst_hbm.at[pl.ds(o0_, CH)], dib[b], lsb[b]).wait()
                pltpu.make_async_copy(
                    ew_hbm.at[pl.ds(o0_, CH)], wib[b], lsb[b]).wait()

            def issue_gather(b):
                pltpu.async_copy(g_ref.at[sib[b]], rvb[b], gsb[b])

            def wait_gather(b):
                pltpu.make_async_copy(g_ref.at[sib[b]], rvb[b], gsb[b]).wait()

            def scale_scatter(b):
                rv, wi, di = rvb[b], wib[b], dib[b]

                def scale(j, _):
                    wv = wi[pl.ds(j * 16, 16)]
                    for k in range(16):
                        w16 = wv.at[jnp.full((16,), k, jnp.int32)].get(
                            mode="promise_in_bounds")
                        r = j * 16 + k
                        rv[r, :] = rv[r, :] * w16
                    return 0
                lax.fori_loop(0, CH // 16, scale, 0)
                pltpu.sync_copy(rv, acc.at[di], add=True)

            def step(i, b):
                wait_gather(b)

                @pl.when(i + 1 < nfull)
                def _():
                    wait_loads(1 - b)
                    issue_gather(1 - b)
                scale_scatter(b)

                @pl.when(i + 2 < nfull)
                def _():
                    issue_loads(i + 2, b)

            issue_loads(0, 0)
            wait_loads(0)
            issue_gather(0)
            if nfull > 1:
                issue_loads(1, 1)

            def pair(p, _):
                step(2 * p, 0)
                step(2 * p + 1, 1)
                return 0
            lax.fori_loop(0, nfull // 2, pair, 0)
            if nfull % 2:
                step(nfull - 1, (nfull - 1) % 2)

            if tail:
                o = pl.multiple_of(base + nfull * CH, 8)
                pltpu.sync_copy(src_hbm.at[pl.ds(o, tail)], src_t)
                pltpu.sync_copy(dst_hbm.at[pl.ds(o, tail)], dst_t)
                pltpu.sync_copy(ew_hbm.at[pl.ds(o, tail)], ew_t)
                pltpu.async_copy(g_ref.at[src_t], rows_t, sem).wait()

                def scale_t(j, _):
                    wv = ew_t[pl.ds(j * 16, 16)]
                    for k in range(16):
                        w16 = wv.at[jnp.full((16,), k, jnp.int32)].get(
                            mode="promise_in_bounds")
                        r = j * 16 + k
                        rows_t[r, :] = rows_t[r, :] * w16
                    return 0
                lax.fori_loop(0, tail // 16, scale_t, 0)
                pltpu.sync_copy(rows_t, acc.at[dst_t], add=True)

        @pl.when(c == 0)
        def _():
            run(g0)

        @pl.when(c == 1)
        def _():
            run(g1)
        plsc.subcore_barrier()

        def flush(out_ref):
            def fcp(i, _):
                r = pl.multiple_of(row0 + i * CH, 8)
                pltpu.sync_copy(acc.at[pl.ds(r, CH), :], zb)
                pltpu.sync_copy(zb, out_ref.at[pl.ds(r, CH), :])
                return 0
            lax.fori_loop(0, nzr, fcp, 0)
            if ztr:
                r = pl.multiple_of(row0 + nzr * CH, 8)
                pltpu.sync_copy(acc.at[pl.ds(r, ztr), :], zb.at[pl.ds(0, ztr), :])
                pltpu.sync_copy(zb.at[pl.ds(0, ztr), :],
                                out_ref.at[pl.ds(r, ztr), :])

        @pl.when(c == 0)
        def _():
            flush(o0)

        @pl.when(c == 1)
        def _():
            flush(o1)

    return seg_kernel


@functools.cache
def _make_take(NI):
    nck, tailc = divmod(NI, CH)
    rounds = -(-nck // (NC * NS))

    scratch = [
        pltpu.VMEM((CH,), jnp.int32),
        pltpu.VMEM((CH,), jnp.float32),
        pltpu.VMEM((max(tailc, 8),), jnp.int32),
        pltpu.VMEM((max(tailc, 8),), jnp.float32),
        pltpu.SemaphoreType.DMA,
    ]

    @functools.partial(
        pl.kernel,
        out_type=jax.ShapeDtypeStruct((NI,), jnp.float32),
        mesh=_mesh(),
        scratch_types=scratch,
        compiler_params=_sc_params(),
    )
    def take_kernel(tab, idx, out, idx_v, val_v, idx_t, val_t, sem):
        c = lax.axis_index("c")
        s = lax.axis_index("s")
        wid = s * NC + c
        for k in range(rounds):
            cid = wid + k * NC * NS

            @pl.when(cid < nck)
            def _(cid=cid):
                o = pl.multiple_of(cid * CH, 8)
                pltpu.sync_copy(idx.at[pl.ds(o, CH)], idx_v)
                pltpu.async_copy(tab.at[idx_v], val_v, sem).wait()
                pltpu.sync_copy(val_v, out.at[pl.ds(o, CH)])
        if tailc:
            @pl.when(wid == 0)
            def _():
                o = pl.multiple_of(nck * CH, 8)
                pltpu.sync_copy(idx.at[pl.ds(o, tailc)], idx_t)
                pltpu.async_copy(tab.at[idx_t], val_t, sem).wait()
                pltpu.sync_copy(val_t, out.at[pl.ds(o, tailc)])

    return take_kernel


@functools.cache
def _make_prep(N, DI, H):
    nb = N // RB

    @functools.partial(
        pl.pallas_call,
        grid=(nb,),
        in_specs=[
            pl.BlockSpec((RB, DI), lambda i: (i, 0)),
            pl.BlockSpec((DI, H), lambda i: (0, 0)),
            pl.BlockSpec((RB, 1), lambda i: (i, 0)),
            pl.BlockSpec((RB, 1), lambda i: (i, 0)),
        ],
        out_specs=[
            pl.BlockSpec((RB, 1), lambda i: (i, 0)),
            pl.BlockSpec((RB, HF), lambda i: (i, 0)),
            pl.BlockSpec((RB, HF), lambda i: (i, 0)),
        ],
        out_shape=[
            jax.ShapeDtypeStruct((N, 1), jnp.float32),
            jax.ShapeDtypeStruct((N, HF), jnp.float32),
            jax.ShapeDtypeStruct((N, HF), jnp.float32),
        ],
    )
    def prep(x_ref, w_ref, d0, d1, dinv_o, g0_o, g1_o):
        deg = d0[...] + d1[...] + 1.0
        dinv = lax.rsqrt(deg)
        g = jnp.dot(x_ref[...], w_ref[...],
                    preferred_element_type=jnp.float32) * dinv
        dinv_o[...] = dinv
        g0_o[...] = g[:, :HF]
        g1_o[...] = g[:, HF:]

    return prep


@functools.cache
def _make_mid(N, H):
    nb = N // RB
    half = pl.BlockSpec((RB, HF), lambda i: (i, 0))

    @functools.partial(
        pl.pallas_call,
        grid=(nb,),
        in_specs=[
            half, half, half, half,
            pl.BlockSpec((RB, 1), lambda i: (i, 0)),
            pl.BlockSpec((1, H), lambda i: (0, 0)),
            pl.BlockSpec((H, H), lambda i: (0, 0)),
        ],
        out_specs=[half, half],
        out_shape=[
            jax.ShapeDtypeStruct((N, HF), jnp.float32),
            jax.ShapeDtypeStruct((N, HF), jnp.float32),
        ],
    )
    def mid(s0, s1, g0, g1, dinv, b, w, o0, o1):
        t = jnp.concatenate([s0[...] + g0[...], s1[...] + g1[...]], axis=1)
        h = jnp.maximum(t * dinv[...] + b[...], 0.0)
        g2 = jnp.dot(h, w[...], preferred_element_type=jnp.float32) * dinv[...]
        o0[...] = g2[:, :HF]
        o1[...] = g2[:, HF:]

    return mid


@functools.cache
def _make_fin(N, H):
    nb = N // RB
    half = pl.BlockSpec((RB, HF), lambda i: (i, 0))

    @functools.partial(
        pl.pallas_call,
        grid=(nb,),
        in_specs=[
            half, half, half, half,
            pl.BlockSpec((RB, 1), lambda i: (i, 0)),
            pl.BlockSpec((1, H), lambda i: (0, 0)),
        ],
        out_specs=pl.BlockSpec((RB, H), lambda i: (i, 0)),
        out_shape=jax.ShapeDtypeStruct((N, H), jnp.float32),
    )
    def fin(s0, s1, g0, g1, dinv, b, z_o):
        t = jnp.concatenate([s0[...] + g0[...], s1[...] + g1[...]], axis=1)
        z_o[...] = jnp.maximum(t * dinv[...] + b[...], 0.0)

    return fin


@functools.cache
def _make_head(NV, NTOT, H, CC, DEC):
    vb = NV // RB
    nb = NTOT // RB
    NK = NTOT - NV

    def cw(shape):
        return pl.BlockSpec(shape, lambda i: (0,) * len(shape))

    @functools.partial(
        pl.pallas_call,
        grid=(nb,),
        in_specs=[
            pl.BlockSpec((RB, H), lambda i: (jnp.minimum(i, vb - 1), 0)),
            pl.BlockSpec((RB, H), lambda i: (i, 0)),
            pl.BlockSpec((RB, CC), lambda i: (i, 0)),
            pl.BlockSpec((1, 1, RB), lambda i: (jnp.minimum(i, vb - 1), 0, 0)),
            cw((CC, CC)), cw((1, CC)), cw((CC, CC)), cw((1, CC)),
            cw((CC, DEC)), cw((1, DEC)), cw((1, DEC)), cw((1, 1)),
            cw((CC, DEC)), cw((1, DEC)), cw((1, DEC)), cw((1, 1)),
            cw((CC, DEC)), cw((1, DEC)), cw((1, DEC)), cw((1, 1)),
        ],
        out_specs=[
            pl.BlockSpec((RB, CC), lambda i: (i, 0)),
            pl.BlockSpec((RB, CC), lambda i: (i, 0)),
            pl.BlockSpec((RB, 1), lambda i: (jnp.minimum(i, vb - 1), 0)),
            pl.BlockSpec((RB, 1), lambda i: (jnp.maximum(i - vb, 0), 0)),
            pl.BlockSpec((RB, 1), lambda i: (jnp.minimum(i, vb - 1), 0)),
            pl.BlockSpec((NG, CC), lambda i: (0, 0)),
            pl.BlockSpec((NG, CC), lambda i: (0, 0)),
        ],
        out_shape=[
            jax.ShapeDtypeStruct((NTOT, CC), jnp.float32),
            jax.ShapeDtypeStruct((NTOT, CC), jnp.float32),
            jax.ShapeDtypeStruct((NV, 1), jnp.float32),
            jax.ShapeDtypeStruct((NK, 1), jnp.float32),
            jax.ShapeDtypeStruct((NV, 1), jnp.float32),
            jax.ShapeDtypeStruct((NG, CC), jnp.float32),
            jax.ShapeDtypeStruct((NG, CC), jnp.float32),
        ],
    )
    def head(zo, zf, eps, bv, muw, mub, lvw, lvb,
             dxw1, dxb1, dxw2, dxb2, dkw1, dkb1, dkw2, dkb2,
             diw1, dib1, diw2, dib2,
             zmu_o, zlv_o, xh_o, pk_o, ig_o, ps_o, pn_o):
        i = pl.program_id(0)
        isv = i < vb
        f = zf[...]
        c0 = jnp.where(isv, zo[...], f)
        c1 = jnp.where(isv, f, jnp.zeros_like(f))
        z = jnp.concatenate([c0, c1], axis=1)
        mu = jnp.dot(z, muw[...], preferred_element_type=jnp.float32) + mub[...]
        lv = jnp.dot(z, lvw[...], preferred_element_type=jnp.float32) + lvb[...]
        zmu_o[...] = mu
        zlv_o[...] = lv
        zz = mu + jnp.exp(0.5 * lv) * eps[...]

        @pl.when(i == 0)
        def _():
            ps_o[...] = jnp.zeros((NG, CC), jnp.float32)
            pn_o[...] = jnp.zeros((NG, CC), jnp.float32)

        @pl.when(isv)
        def _():
            h = jnp.maximum(
                jnp.dot(zz, dxw1[...], preferred_element_type=jnp.float32)
                + dxb1[...], 0.0)
            xh_o[...] = jnp.sum(h * dxw2[...], axis=1, keepdims=True) + dxb2[0, 0]
            hi = jnp.maximum(
                jnp.dot(zz, diw1[...], preferred_element_type=jnp.float32)
                + dib1[...], 0.0)
            logit = jnp.sum(hi * diw2[...], axis=1, keepdims=True) + dib2[0, 0]
            ig_o[...] = 1.0 / (1.0 + jnp.exp(-logit))
            grp = bv[0, 0, :]
            m = (lax.broadcasted_iota(jnp.int32, (NG, RB), 0)
                 == grp[None, :]).astype(jnp.float32)
            ps_o[...] += jnp.dot(m, zz, preferred_element_type=jnp.float32)
            pn_o[...] += jnp.dot(m, jnp.ones((RB, CC), jnp.float32),
                                 preferred_element_type=jnp.float32)

        @pl.when(jnp.logical_not(isv))
        def _():
            hk = jnp.maximum(
                jnp.dot(zz, dkw1[...], preferred_element_type=jnp.float32)
                + dkb1[...], 0.0)
            pk_o[...] = jnp.sum(hk * dkw2[...], axis=1, keepdims=True) + dkb2[0, 0]

    return head


@functools.cache
def _make_cost(CC, DEC):
    @functools.partial(
        pl.pallas_call,
        grid=(1,),
        in_specs=[
            pl.BlockSpec((NG, CC), lambda i: (0, 0)),
            pl.BlockSpec((NG, CC), lambda i: (0, 0)),
            pl.BlockSpec((CC, DEC), lambda i: (0, 0)),
            pl.BlockSpec((1, DEC), lambda i: (0, 0)),
            pl.BlockSpec((1, DEC), lambda i: (0, 0)),
            pl.BlockSpec((1, 1), lambda i: (0, 0)),
        ],
        out_specs=pl.BlockSpec((NG, 1), lambda i: (0, 0)),
        out_shape=jax.ShapeDtypeStruct((NG, 1), jnp.float32),
    )
    def cost(ps, pn, w1, b1, w2, b2, out):
        pooled = ps[...] / jnp.maximum(pn[...], 1.0)
        h = jnp.maximum(
            jnp.dot(pooled, w1[...], preferred_element_type=jnp.float32)
            + b1[...], 0.0)
        out[...] = jnp.sum(h * w2[...], axis=1, keepdims=True) + b2[0, 0]

    return cost


def _encode(x, src, dst, ew, W1, b1, W2, b2):
    n = x.shape[0]
    e = ew.shape[0]
    h = W1.shape[1]
    d0, d1 = _make_deg(e, n)(dst, ew)
    dinv, g0, g1 = _make_prep(n, x.shape[1], h)(
        x, W1, d0[:n, :1], d1[:n, :1])
    s0, s1 = _make_segsum(e, n)(g0, g1, src, dst, ew)
    h0, h1 = _make_mid(n, h)(
        s0[:n], s1[:n], g0, g1, dinv, b1.reshape(1, -1), W2)
    t0, t1 = _make_segsum(e, n)(h0, h1, src, dst, ew)
    return _make_fin(n, h)(t0[:n], t1[:n], h0, h1, dinv, b2.reshape(1, -1))


def kernel(x_obj, edge_index_obj, edge_weight_obj, x_feas, edge_index_feas,
           edge_weight_feas, batch_var, binary_idx, eps, params):
    p = params
    n_var = batch_var.shape[0]
    n_tot = x_feas.shape[0]
    cc = eps.shape[1]
    dec = p["dx_W1"].shape[1]

    z_obj = _encode(x_obj, edge_index_obj[0], edge_index_obj[1],
                    edge_weight_obj, p["obj_W1"], p["obj_b1"],
                    p["obj_W2"], p["obj_b2"])
    zf = _encode(x_feas, edge_index_feas[0], edge_index_feas[1],
                 edge_weight_feas, p["cons_W1"], p["cons_b1"],
                 p["cons_W2"], p["cons_b2"])

    bv3 = batch_var.reshape(n_var // RB, 1, RB)
    zmu, zlv, xh, pk, ig, ps, pn = _make_head(
        n_var, n_tot, z_obj.shape[1], cc, dec)(
        z_obj, zf, eps, bv3,
        p["mu_W"], p["mu_b"].reshape(1, -1),
        p["lv_W"], p["lv_b"].reshape(1, -1),
        p["dx_W1"], p["dx_b1"].reshape(1, -1),
        p["dx_W2"].reshape(1, -1), p["dx_b2"].reshape(1, 1),
        p["dk_W1"], p["dk_b1"].reshape(1, -1),
        p["dk_W2"].reshape(1, -1), p["dk_b2"].reshape(1, 1),
        p["di_W1"], p["di_b1"].reshape(1, -1),
        p["di_W2"].reshape(1, -1), p["di_b2"].reshape(1, 1))

    cost = _make_cost(cc, dec)(
        ps, pn, p["dc_W1"], p["dc_b1"].reshape(1, -1),
        p["dc_W2"].reshape(1, -1), p["dc_b2"].reshape(1, 1))
    pint = _make_take(binary_idx.shape[0])(ig.reshape(-1), binary_idx)
    return (xh.reshape(-1), cost.reshape(-1), pk.reshape(-1), pint, zmu, zlv)


# speedup vs baseline: 21.7258x; 1.1987x over previous
import functools

import jax
import jax.numpy as jnp
from jax import lax
from jax.experimental import pallas as pl
from jax.experimental.pallas import tpu as pltpu
from jax.experimental.pallas import tpu_sc as plsc

NC = 2
NS = 16
CH = 128
RB = 800
NG = 16
HF = 16


def _align8(v):
    return -(-v // 8) * 8


def _mesh():
    return plsc.VectorSubcoreMesh(core_axis_name="c", subcore_axis_name="s")


def _sc_params():
    return pltpu.CompilerParams(use_tc_tiling_on_sc=False)


@functools.cache
def _make_deg(E, N):
    ept = E // (NC * NS)
    assert ept * NC * NS == E and ept % 16 == 0
    nfull, tail = divmod(ept, CH)
    assert tail % 16 == 0
    rpt = _align8(-(-N // NS))
    npad = rpt * NS
    nzr, ztr = divmod(rpt, CH)

    scratch = [
        pltpu.VMEM((CH,), jnp.int32),
        pltpu.VMEM((CH,), jnp.float32),
        pltpu.VMEM((CH, HF), jnp.float32),
        pltpu.VMEM((CH,), jnp.int32),
        pltpu.VMEM((CH,), jnp.float32),
        pltpu.VMEM((CH, HF), jnp.float32),
        pltpu.VMEM((max(tail, 16),), jnp.int32),
        pltpu.VMEM((max(tail, 16),), jnp.float32),
        pltpu.VMEM((max(tail, 16), HF), jnp.float32),
        pltpu.VMEM((CH, HF), jnp.float32),
        pltpu.VMEM_SHARED((npad, HF), jnp.float32),
        pltpu.SemaphoreType.DMA,
        pltpu.SemaphoreType.DMA,
        pltpu.SemaphoreType.DMA,
    ]

    @functools.partial(
        pl.kernel,
        out_type=[jax.ShapeDtypeStruct((npad, HF), jnp.float32)] * NC,
        mesh=_mesh(),
        scratch_types=scratch,
        compiler_params=_sc_params(),
    )
    def deg_kernel(dst_hbm, ew_hbm, out0, out1, di0, wi0, rv0, di1, wi1, rv1,
                   dst_t, ew_t, rows_t, zb, acc, ls0, ls1, sem):
        c = lax.axis_index("c")
        s = lax.axis_index("s")
        wid = s * NC + c
        dib, wib, rvb, lsb = [di0, di1], [wi0, wi1], [rv0, rv1], [ls0, ls1]

        def zloop(i, _):
            zb[i, :] = jnp.zeros((HF,), jnp.float32)
            return 0
        lax.fori_loop(0, CH, zloop, 0)

        row0 = pl.multiple_of(s * rpt, 8)

        def zcp(i, _):
            r = pl.multiple_of(row0 + i * CH, 8)
            pltpu.sync_copy(zb, acc.at[pl.ds(r, CH), :])
            return 0
        lax.fori_loop(0, nzr, zcp, 0)
        if ztr:
            r = pl.multiple_of(row0 + nzr * CH, 8)
            pltpu.sync_copy(zb.at[pl.ds(0, ztr), :], acc.at[pl.ds(r, ztr), :])
        plsc.subcore_barrier()

        def oloop(i, _):
            zb[i, :] = jnp.ones((HF,), jnp.float32)
            return 0
        lax.fori_loop(0, CH, oloop, 0)

        base = wid * ept

        def issue_loads(i, b):
            o = pl.multiple_of(base + i * CH, 8)
            pltpu.async_copy(dst_hbm.at[pl.ds(o, CH)], dib[b], lsb[b])
            pltpu.async_copy(ew_hbm.at[pl.ds(o, CH)], wib[b], lsb[b])

        def wait_loads(b):
            o0_ = pl.multiple_of(base, 8)
            pltpu.make_async_copy(
                dst_hbm.at[pl.ds(o0_, CH)], dib[b], lsb[b]).wait()
            pltpu.make_async_copy(
                ew_hbm.at[pl.ds(o0_, CH)], wib[b], lsb[b]).wait()

        def fill_scatter(b, n, di, wi, rv):
            def fill(j, _):
                wv = wi[pl.ds(j * 16, 16)]
                for k in range(16):
                    w16 = wv.at[jnp.full((16,), k, jnp.int32)].get(
                        mode="promise_in_bounds")
                    r = j * 16 + k
                    rv[r, :] = zb[r, :] * w16
                return 0
            lax.fori_loop(0, n // 16, fill, 0)
            pltpu.sync_copy(rv, acc.at[di], add=True)

        def step(i, b):
            wait_loads(b)
            fill_scatter(b, CH, dib[b], wib[b], rvb[b])

            @pl.when(i + 2 < nfull)
            def _():
                issue_loads(i + 2, b)

        issue_loads(0, 0)
        if nfull > 1:
            issue_loads(1, 1)

        def pair(p, _):
            step(2 * p, 0)
            step(2 * p + 1, 1)
            return 0
        lax.fori_loop(0, nfull // 2, pair, 0)
        if nfull % 2:
            step(nfull - 1, (nfull - 1) % 2)

        if tail:
            o = pl.multiple_of(base + nfull * CH, 8)
            pltpu.sync_copy(dst_hbm.at[pl.ds(o, tail)], dst_t)
            pltpu.sync_copy(ew_hbm.at[pl.ds(o, tail)], ew_t)
            fill_scatter(0, tail, dst_t, ew_t, rows_t)
        plsc.subcore_barrier()

        def flush(out_ref):
            def fcp(i, _):
                r = pl.multiple_of(row0 + i * CH, 8)
                pltpu.sync_copy(acc.at[pl.ds(r, CH), :], zb)
                pltpu.sync_copy(zb, out_ref.at[pl.ds(r, CH), :])
                return 0
            lax.fori_loop(0, nzr, fcp, 0)
            if ztr:
                r = pl.multiple_of(row0 + nzr * CH, 8)
                pltpu.sync_copy(acc.at[pl.ds(r, ztr), :], zb.at[pl.ds(0, ztr), :])
                pltpu.sync_copy(zb.at[pl.ds(0, ztr), :],
                                out_ref.at[pl.ds(r, ztr), :])

        @pl.when(c == 0)
        def _():
            flush(out0)

        @pl.when(c == 1)
        def _():
            flush(out1)

    return deg_kernel


@functools.cache
def _make_segsum(E, N):
    ept = E // NS
    assert ept * NS == E and ept % 8 == 0
    nfull, tail = divmod(ept, CH)
    rpt = _align8(-(-N // NS))
    npad = rpt * NS
    nzr, ztr = divmod(rpt, CH)

    scratch = [
        pltpu.VMEM((CH,), jnp.int32),
        pltpu.VMEM((CH,), jnp.int32),
        pltpu.VMEM((CH,), jnp.float32),
        pltpu.VMEM((CH, HF), jnp.float32),
        pltpu.VMEM((CH,), jnp.int32),
        pltpu.VMEM((CH,), jnp.int32),
        pltpu.VMEM((CH,), jnp.float32),
        pltpu.VMEM((CH, HF), jnp.float32),
        pltpu.VMEM((max(tail, 16),), jnp.int32),
        pltpu.VMEM((max(tail, 16),), jnp.int32),
        pltpu.VMEM((max(tail, 16),), jnp.float32),
        pltpu.VMEM((max(tail, 16), HF), jnp.float32),
        pltpu.VMEM((CH, HF), jnp.float32),
        pltpu.VMEM_SHARED((npad, HF), jnp.float32),
        pltpu.SemaphoreType.DMA,
        pltpu.SemaphoreType.DMA,
        pltpu.SemaphoreType.DMA,
        pltpu.SemaphoreType.DMA,
        pltpu.SemaphoreType.DMA,
    ]

    @functools.partial(
        pl.kernel,
        out_type=[jax.ShapeDtypeStruct((npad, HF), jnp.float32)] * NC,
        mesh=_mesh(),
        scratch_types=scratch,
        compiler_params=_sc_params(),
    )
    def seg_kernel(g0, g1, src_hbm, dst_hbm, ew_hbm, o0, o1,
                   si0, di0, wi0, rv0, si1, di1, wi1, rv1,
                   src_t, dst_t, ew_t, rows_t,
                   zb, acc, ls0, ls1, gs0, gs1, sem):
        c = lax.axis_index("c")
        s = lax.axis_index("s")
        sib, dib, wib, rvb = [si0, si1], [di0, di1], [wi0, wi1], [rv0, rv1]
        lsb, gsb = [ls0, ls1], [gs0, gs1]

        def zloop(i, _):
            zb[i, :] = jnp.zeros((HF,), jnp.float32)
            return 0
        lax.fori_loop(0, CH, zloop, 0)

        row0 = pl.multiple_of(s * rpt, 8)

        def zcp(i, _):
            r = pl.multiple_of(row0 + i * CH, 8)
            pltpu.sync_copy(zb, acc.at[pl.ds(r, CH), :])
            return 0
        lax.fori_loop(0, nzr, zcp, 0)
        if ztr:
            r = pl.multiple_of(row0 + nzr * CH, 8)
            pltpu.sync_copy(zb.at[pl.ds(0, ztr), :], acc.at[pl.ds(r, ztr), :])
        plsc.subcore_barrier()

        base = s * ept

        def run(g_ref):
            def issue_loads(i, b):
                o = pl.multiple_of(base + i * CH, 8)
                pltpu.async_copy(src_hbm.at[pl.ds(o, CH)], sib[b], lsb[b])
                pltpu.async_copy(dst_hbm.at[pl.ds(o, CH)], dib[b], lsb[b])
                pltpu.async_copy(ew_hbm.at[pl.ds(o, CH)], wib[b], lsb[b])

            def wait_loads(b):
                o0_ = pl.multiple_of(base, 8)
                pltpu.make_async_copy(
                    src_hbm.at[pl.ds(o0_, CH)], sib[b], lsb[b]).wait()
                pltpu.make_async_copy(
                    dst_hbm.at[pl.ds(o0_, CH)], dib[b], lsb[b]).wait()
                pltpu.make_async_copy(
                    ew_hbm.at[pl.ds(o0_, CH)], wib[b], lsb[b]).wait()

            def issue_gather(b):
                pltpu.async_copy(g_ref.at[sib[b]], rvb[b], gsb[b])

            def wait_gather(b):
                pltpu.make_async_copy(g_ref.at[sib[b]], rvb[b], gsb[b]).wait()

            def scale_scatter(b):
                rv, wi, di = rvb[b], wib[b], dib[b]

                def scale(j, _):
                    wv = wi[pl.ds(j * 16, 16)]
                    for k in range(16):
                        w16 = wv.at[jnp.full((16,), k, jnp.int32)].get(
                            mode="promise_in_bounds")
                        r = j * 16 + k
                        rv[r, :] = rv[r, :] * w16
                    return 0
                lax.fori_loop(0, CH // 16, scale, 0)
                pltpu.sync_copy(rv, acc.at[di], add=True)

            def step(i, b):
                wait_gather(b)

                @pl.when(i + 1 < nfull)
                def _():
                    wait_loads(1 - b)
                    issue_gather(1 - b)
                scale_scatter(b)

                @pl.when(i + 2 < nfull)
                def _():
                    issue_loads(i + 2, b)

            issue_loads(0, 0)
            wait_loads(0)
            issue_gather(0)
            if nfull > 1:
                issue_loads(1, 1)

            def pair(p, _):
                step(2 * p, 0)
                step(2 * p + 1, 1)
                return 0
            lax.fori_loop(0, nfull // 2, pair, 0)
            if nfull % 2:
                step(nfull - 1, (nfull - 1) % 2)

            if tail:
                o = pl.multiple_of(base + nfull * CH, 8)
                pltpu.sync_copy(src_hbm.at[pl.ds(o, tail)], src_t)
                pltpu.sync_copy(dst_hbm.at[pl.ds(o, tail)], dst_t)
                pltpu.sync_copy(ew_hbm.at[pl.ds(o, tail)], ew_t)
                pltpu.async_copy(g_ref.at[src_t], rows_t, sem).wait()

                def scale_t(j, _):
                    wv = ew_t[pl.ds(j * 16, 16)]
                    for k in range(16):
                        w16 = wv.at[jnp.full((16,), k, jnp.int32)].get(
                            mode="promise_in_bounds")
                        r = j * 16 + k
                        rows_t[r, :] = rows_t[r, :] * w16
                    return 0
                lax.fori_loop(0, tail // 16, scale_t, 0)
                pltpu.sync_copy(rows_t, acc.at[dst_t], add=True)

        @pl.when(c == 0)
        def _():
            run(g0)

        @pl.when(c == 1)
        def _():
            run(g1)
        plsc.subcore_barrier()

        def flush(out_ref):
            def fcp(i, _):
                r = pl.multiple_of(row0 + i * CH, 8)
                pltpu.sync_copy(acc.at[pl.ds(r, CH), :], zb)
                pltpu.sync_copy(zb, out_ref.at[pl.ds(r, CH), :])
                return 0
            lax.fori_loop(0, nzr, fcp, 0)
            if ztr:
                r = pl.multiple_of(row0 + nzr * CH, 8)
                pltpu.sync_copy(acc.at[pl.ds(r, ztr), :], zb.at[pl.ds(0, ztr), :])
                pltpu.sync_copy(zb.at[pl.ds(0, ztr), :],
                                out_ref.at[pl.ds(r, ztr), :])

        @pl.when(c == 0)
        def _():
            flush(o0)

        @pl.when(c == 1)
        def _():
            flush(o1)

    return seg_kernel


@functools.cache
def _make_take(NI):
    nck, tailc = divmod(NI, CH)
    rounds = -(-nck // (NC * NS))

    scratch = [
        pltpu.VMEM((CH,), jnp.int32),
        pltpu.VMEM((CH,), jnp.float32),
        pltpu.VMEM((max(tailc, 8),), jnp.int32),
        pltpu.VMEM((max(tailc, 8),), jnp.float32),
        pltpu.SemaphoreType.DMA,
    ]

    @functools.partial(
        pl.kernel,
        out_type=jax.ShapeDtypeStruct((NI,), jnp.float32),
        mesh=_mesh(),
        scratch_types=scratch,
        compiler_params=_sc_params(),
    )
    def take_kernel(tab, idx, out, idx_v, val_v, idx_t, val_t, sem):
        c = lax.axis_index("c")
        s = lax.axis_index("s")
        wid = s * NC + c
        for k in range(rounds):
            cid = wid + k * NC * NS

            @pl.when(cid < nck)
            def _(cid=cid):
                o = pl.multiple_of(cid * CH, 8)
                pltpu.sync_copy(idx.at[pl.ds(o, CH)], idx_v)
                pltpu.async_copy(tab.at[idx_v], val_v, sem).wait()
                pltpu.sync_copy(val_v, out.at[pl.ds(o, CH)])
        if tailc:
            @pl.when(wid == 0)
            def _():
                o = pl.multiple_of(nck * CH, 8)
                pltpu.sync_copy(idx.at[pl.ds(o, tailc)], idx_t)
                pltpu.async_copy(tab.at[idx_t], val_t, sem).wait()
                pltpu.sync_copy(val_t, out.at[pl.ds(o, tailc)])

    return take_kernel


@functools.cache
def _make_segsum_split(E, N):
    ept = E // (NC * NS)
    assert ept * NC * NS == E and ept % 16 == 0
    nfull, tail = divmod(ept, CH)
    assert tail % 16 == 0
    rpt = _align8(-(-N // NS))
    npad = rpt * NS
    nzr, ztr = divmod(rpt, CH)

    scratch = [
        pltpu.VMEM((CH,), jnp.int32),
        pltpu.VMEM((CH,), jnp.int32),
        pltpu.VMEM((CH,), jnp.float32),
        pltpu.VMEM((CH, HF), jnp.float32),
        pltpu.VMEM((CH,), jnp.int32),
        pltpu.VMEM((CH,), jnp.int32),
        pltpu.VMEM((CH,), jnp.float32),
        pltpu.VMEM((CH, HF), jnp.float32),
        pltpu.VMEM((max(tail, 16),), jnp.int32),
        pltpu.VMEM((max(tail, 16),), jnp.int32),
        pltpu.VMEM((max(tail, 16),), jnp.float32),
        pltpu.VMEM((max(tail, 16), HF), jnp.float32),
        pltpu.VMEM((CH, HF), jnp.float32),
        pltpu.VMEM_SHARED((npad, HF), jnp.float32),
        pltpu.SemaphoreType.DMA,
        pltpu.SemaphoreType.DMA,
        pltpu.SemaphoreType.DMA,
        pltpu.SemaphoreType.DMA,
        pltpu.SemaphoreType.DMA,
    ]

    @functools.partial(
        pl.kernel,
        out_type=[jax.ShapeDtypeStruct((npad, HF), jnp.float32)] * NC,
        mesh=_mesh(),
        scratch_types=scratch,
        compiler_params=_sc_params(),
    )
    def seg_kernel(g_hbm, src_hbm, dst_hbm, ew_hbm, o0, o1,
                   si0, di0, wi0, rv0, si1, di1, wi1, rv1,
                   src_t, dst_t, ew_t, rows_t,
                   zb, acc, ls0, ls1, gs0, gs1, sem):
        c = lax.axis_index("c")
        s = lax.axis_index("s")
        wid = s * NC + c
        sib, dib, wib, rvb = [si0, si1], [di0, di1], [wi0, wi1], [rv0, rv1]
        lsb, gsb = [ls0, ls1], [gs0, gs1]

        def zloop(i, _):
            zb[i, :] = jnp.zeros((HF,), jnp.float32)
            return 0
        lax.fori_loop(0, CH, zloop, 0)

        row0 = pl.multiple_of(s * rpt, 8)

        def zcp(i, _):
            r = pl.multiple_of(row0 + i * CH, 8)
            pltpu.sync_copy(zb, acc.at[pl.ds(r, CH), :])
            return 0
        lax.fori_loop(0, nzr, zcp, 0)
        if ztr:
            r = pl.multiple_of(row0 + nzr * CH, 8)
            pltpu.sync_copy(zb.at[pl.ds(0, ztr), :], acc.at[pl.ds(r, ztr), :])
        plsc.subcore_barrier()

        base = wid * ept

        def issue_loads(i, b):
            o = pl.multiple_of(base + i * CH, 8)
            pltpu.async_copy(src_hbm.at[pl.ds(o, CH)], sib[b], lsb[b])
            pltpu.async_copy(dst_hbm.at[pl.ds(o, CH)], dib[b], lsb[b])
            pltpu.async_copy(ew_hbm.at[pl.ds(o, CH)], wib[b], lsb[b])

        def wait_loads(b):
            o0_ = pl.multiple_of(base, 8)
            pltpu.make_async_copy(
                src_hbm.at[pl.ds(o0_, CH)], sib[b], lsb[b]).wait()
            pltpu.make_async_copy(
                dst_hbm.at[pl.ds(o0_, CH)], dib[b], lsb[b]).wait()
            pltpu.make_async_copy(
                ew_hbm.at[pl.ds(o0_, CH)], wib[b], lsb[b]).wait()

        def issue_gather(b):
            pltpu.async_copy(g_hbm.at[sib[b]], rvb[b], gsb[b])

        def wait_gather(b):
            pltpu.make_async_copy(g_hbm.at[sib[b]], rvb[b], gsb[b]).wait()

        def scale_scatter(b):
            rv, wi, di = rvb[b], wib[b], dib[b]

            def scale(j, _):
                wv = wi[pl.ds(j * 16, 16)]
                for k in range(16):
                    w16 = wv.at[jnp.full((16,), k, jnp.int32)].get(
                        mode="promise_in_bounds")
                    r = j * 16 + k
                    rv[r, :] = rv[r, :] * w16
                return 0
            lax.fori_loop(0, CH // 16, scale, 0)
            pltpu.sync_copy(rv, acc.at[di], add=True)

        def step(i, b):
            wait_gather(b)

            @pl.when(i + 1 < nfull)
            def _():
                wait_loads(1 - b)
                issue_gather(1 - b)
            scale_scatter(b)

            @pl.when(i + 2 < nfull)
            def _():
                issue_loads(i + 2, b)

        issue_loads(0, 0)
        wait_loads(0)
        issue_gather(0)
        if nfull > 1:
            issue_loads(1, 1)

        def pair(p, _):
            step(2 * p, 0)
            step(2 * p + 1, 1)
            return 0
        lax.fori_loop(0, nfull // 2, pair, 0)
        if nfull % 2:
            step(nfull - 1, (nfull - 1) % 2)

        if tail:
            o = pl.multiple_of(base + nfull * CH, 8)
            pltpu.sync_copy(src_hbm.at[pl.ds(o, tail)], src_t)
            pltpu.sync_copy(dst_hbm.at[pl.ds(o, tail)], dst_t)
            pltpu.sync_copy(ew_hbm.at[pl.ds(o, tail)], ew_t)
            pltpu.async_copy(g_hbm.at[src_t], rows_t, sem).wait()

            def scale_t(j, _):
                wv = ew_t[pl.ds(j * 16, 16)]
                for k in range(16):
                    w16 = wv.at[jnp.full((16,), k, jnp.int32)].get(
                        mode="promise_in_bounds")
                    r = j * 16 + k
                    rows_t[r, :] = rows_t[r, :] * w16
                return 0
            lax.fori_loop(0, tail // 16, scale_t, 0)
            pltpu.sync_copy(rows_t, acc.at[dst_t], add=True)
        plsc.subcore_barrier()

        def flush(out_ref):
            def fcp(i, _):
                r = pl.multiple_of(row0 + i * CH, 8)
                pltpu.sync_copy(acc.at[pl.ds(r, CH), :], zb)
                pltpu.sync_copy(zb, out_ref.at[pl.ds(r, CH), :])
                return 0
            lax.fori_loop(0, nzr, fcp, 0)
            if ztr:
                r = pl.multiple_of(row0 + nzr * CH, 8)
                pltpu.sync_copy(acc.at[pl.ds(r, ztr), :], zb.at[pl.ds(0, ztr), :])
                pltpu.sync_copy(zb.at[pl.ds(0, ztr), :],
                                out_ref.at[pl.ds(r, ztr), :])

        @pl.when(c == 0)
        def _():
            flush(o0)

        @pl.when(c == 1)
        def _():
            flush(o1)

    return seg_kernel


@functools.cache
def _make_prep(N, DI):
    nb = N // RB

    @functools.partial(
        pl.pallas_call,
        grid=(nb,),
        in_specs=[
            pl.BlockSpec((RB, DI), lambda i: (i, 0)),
            pl.BlockSpec((RB, 1), lambda i: (i, 0)),
            pl.BlockSpec((RB, 1), lambda i: (i, 0)),
        ],
        out_specs=[
            pl.BlockSpec((RB, 1), lambda i: (i, 0)),
            pl.BlockSpec((RB, HF), lambda i: (i, 0)),
        ],
        out_shape=[
            jax.ShapeDtypeStruct((N, 1), jnp.float32),
            jax.ShapeDtypeStruct((N, HF), jnp.float32),
        ],
    )
    def prep(x_ref, d0, d1, dinv_o, y_o):
        deg = d0[...] + d1[...] + 1.0
        dinv = lax.rsqrt(deg)
        dinv_o[...] = dinv
        y_o[...] = jnp.concatenate(
            [x_ref[...] * dinv, jnp.zeros((RB, HF - DI), jnp.float32)], axis=1)

    return prep


@functools.cache
def _make_mid(N, DI, H):
    nb = N // RB
    half = pl.BlockSpec((RB, HF), lambda i: (i, 0))

    @functools.partial(
        pl.pallas_call,
        grid=(nb,),
        in_specs=[
            half, half, half,
            pl.BlockSpec((RB, 1), lambda i: (i, 0)),
            pl.BlockSpec((DI, H), lambda i: (0, 0)),
            pl.BlockSpec((1, H), lambda i: (0, 0)),
            pl.BlockSpec((H, H), lambda i: (0, 0)),
        ],
        out_specs=[half, half],
        out_shape=[
            jax.ShapeDtypeStruct((N, HF), jnp.float32),
            jax.ShapeDtypeStruct((N, HF), jnp.float32),
        ],
    )
    def mid(sa, sb, y, dinv, w1, b, w2, o0, o1):
        t = (sa[...] + sb[...] + y[...])[:, :DI]
        h = jnp.maximum(
            jnp.dot(t, w1[...], preferred_element_type=jnp.float32)
            * dinv[...] + b[...], 0.0)
        g2 = jnp.dot(h, w2[...], preferred_element_type=jnp.float32) * dinv[...]
        o0[...] = g2[:, :HF]
        o1[...] = g2[:, HF:]

    return mid


@functools.cache
def _make_fin(N, H):
    nb = N // RB
    half = pl.BlockSpec((RB, HF), lambda i: (i, 0))

    @functools.partial(
        pl.pallas_call,
        grid=(nb,),
        in_specs=[
            half, half, half, half,
            pl.BlockSpec((RB, 1), lambda i: (i, 0)),
            pl.BlockSpec((1, H), lambda i: (0, 0)),
        ],
        out_specs=pl.BlockSpec((RB, H), lambda i: (i, 0)),
        out_shape=jax.ShapeDtypeStruct((N, H), jnp.float32),
    )
    def fin(s0, s1, g0, g1, dinv, b, z_o):
        t = jnp.concatenate([s0[...] + g0[...], s1[...] + g1[...]], axis=1)
        z_o[...] = jnp.maximum(t * dinv[...] + b[...], 0.0)

    return fin


@functools.cache
def _make_head(NV, NTOT, H, CC, DEC):
    vb = NV // RB
    nb = NTOT // RB
    NK = NTOT - NV

    def cw(shape):
        return pl.BlockSpec(shape, lambda i: (0,) * len(shape))

    @functools.partial(
        pl.pallas_call,
        grid=(nb,),
        in_specs=[
            pl.BlockSpec((RB, H), lambda i: (jnp.minimum(i, vb - 1), 0)),
            pl.BlockSpec((RB, H), lambda i: (i, 0)),
            pl.BlockSpec((RB, CC), lambda i: (i, 0)),
            pl.BlockSpec((1, 1, RB), lambda i: (jnp.minimum(i, vb - 1), 0, 0)),
            cw((CC, CC)), cw((1, CC)), cw((CC, CC)), cw((1, CC)),
            cw((CC, DEC)), cw((1, DEC)), cw((1, DEC)), cw((1, 1)),
            cw((CC, DEC)), cw((1, DEC)), cw((1, DEC)), cw((1, 1)),
            cw((CC, DEC)), cw((1, DEC)), cw((1, DEC)), cw((1, 1)),
        ],
        out_specs=[
            pl.BlockSpec((RB, CC), lambda i: (i, 0)),
            pl.BlockSpec((RB, CC), lambda i: (i, 0)),
            pl.BlockSpec((RB, 1), lambda i: (jnp.minimum(i, vb - 1), 0)),
            pl.BlockSpec((RB, 1), lambda i: (jnp.maximum(i - vb, 0), 0)),
            pl.BlockSpec((RB, 1), lambda i: (jnp.minimum(i, vb - 1), 0)),
            pl.BlockSpec((NG, CC), lambda i: (0, 0)),
            pl.BlockSpec((NG, CC), lambda i: (0, 0)),
        ],
        out_shape=[
            jax.ShapeDtypeStruct((NTOT, CC), jnp.float32),
            jax.ShapeDtypeStruct((NTOT, CC), jnp.float32),
            jax.ShapeDtypeStruct((NV, 1), jnp.float32),
            jax.ShapeDtypeStruct((NK, 1), jnp.float32),
            jax.ShapeDtypeStruct((NV, 1), jnp.float32),
            jax.ShapeDtypeStruct((NG, CC), jnp.float32),
            jax.ShapeDtypeStruct((NG, CC), jnp.float32),
        ],
    )
    def head(zo, zf, eps, bv, muw, mub, lvw, lvb,
             dxw1, dxb1, dxw2, dxb2, dkw1, dkb1, dkw2, dkb2,
             diw1, dib1, diw2, dib2,
             zmu_o, zlv_o, xh_o, pk_o, ig_o, ps_o, pn_o):
        i = pl.program_id(0)
        isv = i < vb
        f = zf[...]
        c0 = jnp.where(isv, zo[...], f)
        c1 = jnp.where(isv, f, jnp.zeros_like(f))
        z = jnp.concatenate([c0, c1], axis=1)
        mu = jnp.dot(z, muw[...], preferred_element_type=jnp.float32) + mub[...]
        lv = jnp.dot(z, lvw[...], preferred_element_type=jnp.float32) + lvb[...]
        zmu_o[...] = mu
        zlv_o[...] = lv
        zz = mu + jnp.exp(0.5 * lv) * eps[...]

        @pl.when(i == 0)
        def _():
            ps_o[...] = jnp.zeros((NG, CC), jnp.float32)
            pn_o[...] = jnp.zeros((NG, CC), jnp.float32)

        @pl.when(isv)
        def _():
            h = jnp.maximum(
                jnp.dot(zz, dxw1[...], preferred_element_type=jnp.float32)
                + dxb1[...], 0.0)
            xh_o[...] = jnp.sum(h * dxw2[...], axis=1, keepdims=True) + dxb2[0, 0]
            hi = jnp.maximum(
                jnp.dot(zz, diw1[...], preferred_element_type=jnp.float32)
                + dib1[...], 0.0)
            logit = jnp.sum(hi * diw2[...], axis=1, keepdims=True) + dib2[0, 0]
            ig_o[...] = 1.0 / (1.0 + jnp.exp(-logit))
            grp = bv[0, 0, :]
            m = (lax.broadcasted_iota(jnp.int32, (NG, RB), 0)
                 == grp[None, :]).astype(jnp.float32)
            ps_o[...] += jnp.dot(m, zz, preferred_element_type=jnp.float32)
            pn_o[...] += jnp.dot(m, jnp.ones((RB, CC), jnp.float32),
                                 preferred_element_type=jnp.float32)

        @pl.when(jnp.logical_not(isv))
        def _():
            hk = jnp.maximum(
                jnp.dot(zz, dkw1[...], preferred_element_type=jnp.float32)
                + dkb1[...], 0.0)
            pk_o[...] = jnp.sum(hk * dkw2[...], axis=1, keepdims=True) + dkb2[0, 0]

    return head


@functools.cache
def _make_cost(CC, DEC):
    @functools.partial(
        pl.pallas_call,
        grid=(1,),
        in_specs=[
            pl.BlockSpec((NG, CC), lambda i: (0, 0)),
            pl.BlockSpec((NG, CC), lambda i: (0, 0)),
            pl.BlockSpec((CC, DEC), lambda i: (0, 0)),
            pl.BlockSpec((1, DEC), lambda i: (0, 0)),
            pl.BlockSpec((1, DEC), lambda i: (0, 0)),
            pl.BlockSpec((1, 1), lambda i: (0, 0)),
        ],
        out_specs=pl.BlockSpec((NG, 1), lambda i: (0, 0)),
        out_shape=jax.ShapeDtypeStruct((NG, 1), jnp.float32),
    )
    def cost(ps, pn, w1, b1, w2, b2, out):
        pooled = ps[...] / jnp.maximum(pn[...], 1.0)
        h = jnp.maximum(
            jnp.dot(pooled, w1[...], preferred_element_type=jnp.float32)
            + b1[...], 0.0)
        out[...] = jnp.sum(h * w2[...], axis=1, keepdims=True) + b2[0, 0]

    return cost


def _encode(x, src, dst, ew, W1, b1, W2, b2):
    n = x.shape[0]
    e = ew.shape[0]
    h = W1.shape[1]
    d0, d1 = _make_deg(e, n)(dst, ew)
    dinv, y = _make_prep(n, x.shape[1])(x, d0[:n, :1], d1[:n, :1])
    sa, sb = _make_segsum_split(e, n)(y, src, dst, ew)
    h0, h1 = _make_mid(n, x.shape[1], h)(
        sa[:n], sb[:n], y, dinv, W1, b1.reshape(1, -1), W2)
    t0, t1 = _make_segsum(e, n)(h0, h1, src, dst, ew)
    return _make_fin(n, h)(t0[:n], t1[:n], h0, h1, dinv, b2.reshape(1, -1))


def kernel(x_obj, edge_index_obj, edge_weight_obj, x_feas, edge_index_feas,
           edge_weight_feas, batch_var, binary_idx, eps, params):
    p = params
    n_var = batch_var.shape[0]
    n_tot = x_feas.shape[0]
    cc = eps.shape[1]
    dec = p["dx_W1"].shape[1]

    z_obj = _encode(x_obj, edge_index_obj[0], edge_index_obj[1],
                    edge_weight_obj, p["obj_W1"], p["obj_b1"],
                    p["obj_W2"], p["obj_b2"])
    zf = _encode(x_feas, edge_index_feas[0], edge_index_feas[1],
                 edge_weight_feas, p["cons_W1"], p["cons_b1"],
                 p["cons_W2"], p["cons_b2"])

    bv3 = batch_var.reshape(n_var // RB, 1, RB)
    zmu, zlv, xh, pk, ig, ps, pn = _make_head(
        n_var, n_tot, z_obj.shape[1], cc, dec)(
        z_obj, zf, eps, bv3,
        p["mu_W"], p["mu_b"].reshape(1, -1),
        p["lv_W"], p["lv_b"].reshape(1, -1),
        p["dx_W1"], p["dx_b1"].reshape(1, -1),
        p["dx_W2"].reshape(1, -1), p["dx_b2"].reshape(1, 1),
        p["dk_W1"], p["dk_b1"].reshape(1, -1),
        p["dk_W2"].reshape(1, -1), p["dk_b2"].reshape(1, 1),
        p["di_W1"], p["di_b1"].reshape(1, -1),
        p["di_W2"].reshape(1, -1), p["di_b2"].reshape(1, 1))

    cost = _make_cost(cc, dec)(
        ps, pn, p["dc_W1"], p["dc_b1"].reshape(1, -1),
        p["dc_W2"].reshape(1, -1), p["dc_b2"].reshape(1, 1))
    pint = _make_take(binary_idx.shape[0])(ig.reshape(-1), binary_idx)
    return (xh.reshape(-1), cost.reshape(-1), pk.reshape(-1), pint, zmu, zlv)


# speedup vs baseline: 22.3771x; 1.0300x over previous
import functools

import jax
import jax.numpy as jnp
from jax import lax
from jax.experimental import pallas as pl
from jax.experimental.pallas import tpu as pltpu
from jax.experimental.pallas import tpu_sc as plsc

NC = 2
NS = 16
CH = 128
RB = 800
NG = 16
HF = 16


def _align8(v):
    return -(-v // 8) * 8


def _mesh():
    return plsc.VectorSubcoreMesh(core_axis_name="c", subcore_axis_name="s")


def _sc_params():
    return pltpu.CompilerParams(use_tc_tiling_on_sc=False)


@functools.cache
def _make_deg(E, N):
    ept = E // (NC * NS)
    assert ept * NC * NS == E and ept % 16 == 0
    nfull, tail = divmod(ept, CH)
    assert tail % 16 == 0
    rpt = _align8(-(-N // NS))
    npad = rpt * NS
    nzr, ztr = divmod(rpt, CH)

    scratch = [
        pltpu.VMEM((CH,), jnp.int32),
        pltpu.VMEM((CH,), jnp.float32),
        pltpu.VMEM((CH, HF), jnp.float32),
        pltpu.VMEM((CH,), jnp.int32),
        pltpu.VMEM((CH,), jnp.float32),
        pltpu.VMEM((CH, HF), jnp.float32),
        pltpu.VMEM((max(tail, 16),), jnp.int32),
        pltpu.VMEM((max(tail, 16),), jnp.float32),
        pltpu.VMEM((max(tail, 16), HF), jnp.float32),
        pltpu.VMEM((CH, HF), jnp.float32),
        pltpu.VMEM_SHARED((npad, HF), jnp.float32),
        pltpu.SemaphoreType.DMA,
        pltpu.SemaphoreType.DMA,
        pltpu.SemaphoreType.DMA,
    ]

    @functools.partial(
        pl.kernel,
        out_type=[jax.ShapeDtypeStruct((npad, HF), jnp.float32)] * NC,
        mesh=_mesh(),
        scratch_types=scratch,
        compiler_params=_sc_params(),
    )
    def deg_kernel(dst_hbm, ew_hbm, out0, out1, di0, wi0, rv0, di1, wi1, rv1,
                   dst_t, ew_t, rows_t, zb, acc, ls0, ls1, sem):
        c = lax.axis_index("c")
        s = lax.axis_index("s")
        wid = s * NC + c
        dib, wib, rvb, lsb = [di0, di1], [wi0, wi1], [rv0, rv1], [ls0, ls1]

        def zloop(i, _):
            zb[i, :] = jnp.zeros((HF,), jnp.float32)
            return 0
        lax.fori_loop(0, CH, zloop, 0)

        row0 = pl.multiple_of(s * rpt, 8)

        def zcp(i, _):
            r = pl.multiple_of(row0 + i * CH, 8)
            pltpu.sync_copy(zb, acc.at[pl.ds(r, CH), :])
            return 0
        lax.fori_loop(0, nzr, zcp, 0)
        if ztr:
            r = pl.multiple_of(row0 + nzr * CH, 8)
            pltpu.sync_copy(zb.at[pl.ds(0, ztr), :], acc.at[pl.ds(r, ztr), :])
        plsc.subcore_barrier()

        def oloop(i, _):
            zb[i, :] = jnp.ones((HF,), jnp.float32)
            return 0
        lax.fori_loop(0, CH, oloop, 0)

        base = wid * ept

        def issue_loads(i, b):
            o = pl.multiple_of(base + i * CH, 8)
            pltpu.async_copy(dst_hbm.at[pl.ds(o, CH)], dib[b], lsb[b])
            pltpu.async_copy(ew_hbm.at[pl.ds(o, CH)], wib[b], lsb[b])

        def wait_loads(b):
            o0_ = pl.multiple_of(base, 8)
            pltpu.make_async_copy(
                dst_hbm.at[pl.ds(o0_, CH)], dib[b], lsb[b]).wait()
            pltpu.make_async_copy(
                ew_hbm.at[pl.ds(o0_, CH)], wib[b], lsb[b]).wait()

        def fill_scatter(b, n, di, wi, rv):
            def fill(j, _):
                wv = wi[pl.ds(j * 16, 16)]
                for k in range(16):
                    w16 = wv.at[jnp.full((16,), k, jnp.int32)].get(
                        mode="promise_in_bounds")
                    r = j * 16 + k
                    rv[r, :] = zb[r, :] * w16
                return 0
            lax.fori_loop(0, n // 16, fill, 0)
            pltpu.sync_copy(rv, acc.at[di], add=True)

        def step(i, b):
            wait_loads(b)
            fill_scatter(b, CH, dib[b], wib[b], rvb[b])

            @pl.when(i + 2 < nfull)
            def _():
                issue_loads(i + 2, b)

        issue_loads(0, 0)
        if nfull > 1:
            issue_loads(1, 1)

        def pair(p, _):
            step(2 * p, 0)
            step(2 * p + 1, 1)
            return 0
        lax.fori_loop(0, nfull // 2, pair, 0)
        if nfull % 2:
            step(nfull - 1, (nfull - 1) % 2)

        if tail:
            o = pl.multiple_of(base + nfull * CH, 8)
            pltpu.sync_copy(dst_hbm.at[pl.ds(o, tail)], dst_t)
            pltpu.sync_copy(ew_hbm.at[pl.ds(o, tail)], ew_t)
            fill_scatter(0, tail, dst_t, ew_t, rows_t)
        plsc.subcore_barrier()

        def flush(out_ref):
            def fcp(i, _):
                r = pl.multiple_of(row0 + i * CH, 8)
                pltpu.sync_copy(acc.at[pl.ds(r, CH), :], zb)
                pltpu.sync_copy(zb, out_ref.at[pl.ds(r, CH), :])
                return 0
            lax.fori_loop(0, nzr, fcp, 0)
            if ztr:
                r = pl.multiple_of(row0 + nzr * CH, 8)
                pltpu.sync_copy(acc.at[pl.ds(r, ztr), :], zb.at[pl.ds(0, ztr), :])
                pltpu.sync_copy(zb.at[pl.ds(0, ztr), :],
                                out_ref.at[pl.ds(r, ztr), :])

        @pl.when(c == 0)
        def _():
            flush(out0)

        @pl.when(c == 1)
        def _():
            flush(out1)

    return deg_kernel


@functools.cache
def _make_segsum(E, N):
    ept = E // NS
    assert ept * NS == E and ept % 8 == 0
    nfull, tail = divmod(ept, CH)
    rpt = _align8(-(-N // NS))
    npad = rpt * NS
    nzr, ztr = divmod(rpt, CH)

    scratch = [
        pltpu.VMEM((CH,), jnp.int32),
        pltpu.VMEM((CH,), jnp.int32),
        pltpu.VMEM((CH,), jnp.float32),
        pltpu.VMEM((CH, HF), jnp.float32),
        pltpu.VMEM((CH,), jnp.int32),
        pltpu.VMEM((CH,), jnp.int32),
        pltpu.VMEM((CH,), jnp.float32),
        pltpu.VMEM((CH, HF), jnp.float32),
        pltpu.VMEM((CH,), jnp.int32),
        pltpu.VMEM((CH,), jnp.int32),
        pltpu.VMEM((CH,), jnp.float32),
        pltpu.VMEM((CH, HF), jnp.float32),
        pltpu.VMEM((max(tail, 16),), jnp.int32),
        pltpu.VMEM((max(tail, 16),), jnp.int32),
        pltpu.VMEM((max(tail, 16),), jnp.float32),
        pltpu.VMEM((max(tail, 16), HF), jnp.float32),
        pltpu.VMEM((CH, HF), jnp.float32),
        pltpu.VMEM_SHARED((npad, HF), jnp.float32),
        pltpu.SemaphoreType.DMA,
        pltpu.SemaphoreType.DMA,
        pltpu.SemaphoreType.DMA,
        pltpu.SemaphoreType.DMA,
        pltpu.SemaphoreType.DMA,
        pltpu.SemaphoreType.DMA,
        pltpu.SemaphoreType.DMA,
    ]

    @functools.partial(
        pl.kernel,
        out_type=[jax.ShapeDtypeStruct((npad, HF), jnp.float32)] * NC,
        mesh=_mesh(),
        scratch_types=scratch,
        compiler_params=_sc_params(),
    )
    def seg_kernel(g0, g1, src_hbm, dst_hbm, ew_hbm, o0, o1,
                   si0, di0, wi0, rv0, si1, di1, wi1, rv1,
                   si2, di2, wi2, rv2,
                   src_t, dst_t, ew_t, rows_t,
                   zb, acc, ls0, ls1, ls2, gs0, gs1, gs2, sem):
        c = lax.axis_index("c")
        s = lax.axis_index("s")
        sib, dib, wib = [si0, si1, si2], [di0, di1, di2], [wi0, wi1, wi2]
        rvb = [rv0, rv1, rv2]
        lsb, gsb = [ls0, ls1, ls2], [gs0, gs1, gs2]

        def zloop(i, _):
            zb[i, :] = jnp.zeros((HF,), jnp.float32)
            return 0
        lax.fori_loop(0, CH, zloop, 0)

        row0 = pl.multiple_of(s * rpt, 8)

        def zcp(i, _):
            r = pl.multiple_of(row0 + i * CH, 8)
            pltpu.sync_copy(zb, acc.at[pl.ds(r, CH), :])
            return 0
        lax.fori_loop(0, nzr, zcp, 0)
        if ztr:
            r = pl.multiple_of(row0 + nzr * CH, 8)
            pltpu.sync_copy(zb.at[pl.ds(0, ztr), :], acc.at[pl.ds(r, ztr), :])
        plsc.subcore_barrier()

        base = s * ept

        def run(g_ref):
            def issue_loads(i, b):
                o = pl.multiple_of(base + i * CH, 8)
                pltpu.async_copy(src_hbm.at[pl.ds(o, CH)], sib[b], lsb[b])
                pltpu.async_copy(dst_hbm.at[pl.ds(o, CH)], dib[b], lsb[b])
                pltpu.async_copy(ew_hbm.at[pl.ds(o, CH)], wib[b], lsb[b])

            def wait_loads(b):
                o0_ = pl.multiple_of(base, 8)
                pltpu.make_async_copy(
                    src_hbm.at[pl.ds(o0_, CH)], sib[b], lsb[b]).wait()
                pltpu.make_async_copy(
                    dst_hbm.at[pl.ds(o0_, CH)], dib[b], lsb[b]).wait()
                pltpu.make_async_copy(
                    ew_hbm.at[pl.ds(o0_, CH)], wib[b], lsb[b]).wait()

            def issue_gather(b):
                pltpu.async_copy(g_ref.at[sib[b]], rvb[b], gsb[b])

            def wait_gather(b):
                pltpu.make_async_copy(g_ref.at[sib[b]], rvb[b], gsb[b]).wait()

            def scale_scatter(b):
                rv, wi, di = rvb[b], wib[b], dib[b]

                def scale(j, _):
                    wv = wi[pl.ds(j * 16, 16)]
                    for k in range(16):
                        w16 = wv.at[jnp.full((16,), k, jnp.int32)].get(
                            mode="promise_in_bounds")
                        r = j * 16 + k
                        rv[r, :] = rv[r, :] * w16
                    return 0
                lax.fori_loop(0, CH // 16, scale, 0)
                pltpu.sync_copy(rv, acc.at[di], add=True)

            def step(i, b):
                wait_gather(b)

                @pl.when(i + 2 < nfull)
                def _():
                    wait_loads((b + 2) % 3)
                    issue_gather((b + 2) % 3)
                scale_scatter(b)

                @pl.when(i + 3 < nfull)
                def _():
                    issue_loads(i + 3, b)

            issue_loads(0, 0)
            wait_loads(0)
            issue_gather(0)
            if nfull > 1:
                issue_loads(1, 1)
                wait_loads(1)
                issue_gather(1)
            if nfull > 2:
                issue_loads(2, 2)

            def triple(p, _):
                step(3 * p, 0)
                step(3 * p + 1, 1)
                step(3 * p + 2, 2)
                return 0
            lax.fori_loop(0, nfull // 3, triple, 0)
            for r_ in range(nfull % 3):
                i_ = nfull - (nfull % 3) + r_
                step(i_, i_ % 3)

            if tail:
                o = pl.multiple_of(base + nfull * CH, 8)
                pltpu.sync_copy(src_hbm.at[pl.ds(o, tail)], src_t)
                pltpu.sync_copy(dst_hbm.at[pl.ds(o, tail)], dst_t)
                pltpu.sync_copy(ew_hbm.at[pl.ds(o, tail)], ew_t)
                pltpu.async_copy(g_ref.at[src_t], rows_t, sem).wait()

                def scale_t(j, _):
                    wv = ew_t[pl.ds(j * 16, 16)]
                    for k in range(16):
                        w16 = wv.at[jnp.full((16,), k, jnp.int32)].get(
                            mode="promise_in_bounds")
                        r = j * 16 + k
                        rows_t[r, :] = rows_t[r, :] * w16
                    return 0
                lax.fori_loop(0, tail // 16, scale_t, 0)
                pltpu.sync_copy(rows_t, acc.at[dst_t], add=True)

        @pl.when(c == 0)
        def _():
            run(g0)

        @pl.when(c == 1)
        def _():
            run(g1)
        plsc.subcore_barrier()

        def flush(out_ref):
            def fcp(i, _):
                r = pl.multiple_of(row0 + i * CH, 8)
                pltpu.sync_copy(acc.at[pl.ds(r, CH), :], zb)
                pltpu.sync_copy(zb, out_ref.at[pl.ds(r, CH), :])
                return 0
            lax.fori_loop(0, nzr, fcp, 0)
            if ztr:
                r = pl.multiple_of(row0 + nzr * CH, 8)
                pltpu.sync_copy(acc.at[pl.ds(r, ztr), :], zb.at[pl.ds(0, ztr), :])
                pltpu.sync_copy(zb.at[pl.ds(0, ztr), :],
                                out_ref.at[pl.ds(r, ztr), :])

        @pl.when(c == 0)
        def _():
            flush(o0)

        @pl.when(c == 1)
        def _():
            flush(o1)

    return seg_kernel


@functools.cache
def _make_take(NI):
    nck, tailc = divmod(NI, CH)
    rounds = -(-nck // (NC * NS))

    scratch = [
        pltpu.VMEM((CH,), jnp.int32),
        pltpu.VMEM((CH,), jnp.float32),
        pltpu.VMEM((max(tailc, 8),), jnp.int32),
        pltpu.VMEM((max(tailc, 8),), jnp.float32),
        pltpu.SemaphoreType.DMA,
    ]

    @functools.partial(
        pl.kernel,
        out_type=jax.ShapeDtypeStruct((NI,), jnp.float32),
        mesh=_mesh(),
        scratch_types=scratch,
        compiler_params=_sc_params(),
    )
    def take_kernel(tab, idx, out, idx_v, val_v, idx_t, val_t, sem):
        c = lax.axis_index("c")
        s = lax.axis_index("s")
        wid = s * NC + c
        for k in range(rounds):
            cid = wid + k * NC * NS

            @pl.when(cid < nck)
            def _(cid=cid):
                o = pl.multiple_of(cid * CH, 8)
                pltpu.sync_copy(idx.at[pl.ds(o, CH)], idx_v)
                pltpu.async_copy(tab.at[idx_v], val_v, sem).wait()
                pltpu.sync_copy(val_v, out.at[pl.ds(o, CH)])
        if tailc:
            @pl.when(wid == 0)
            def _():
                o = pl.multiple_of(nck * CH, 8)
                pltpu.sync_copy(idx.at[pl.ds(o, tailc)], idx_t)
                pltpu.async_copy(tab.at[idx_t], val_t, sem).wait()
                pltpu.sync_copy(val_t, out.at[pl.ds(o, tailc)])

    return take_kernel


@functools.cache
def _make_segsum_split(E, N):
    ept = E // (NC * NS)
    assert ept * NC * NS == E and ept % 16 == 0
    nfull, tail = divmod(ept, CH)
    assert tail % 16 == 0
    rpt = _align8(-(-N // NS))
    npad = rpt * NS
    nzr, ztr = divmod(rpt, CH)

    scratch = [
        pltpu.VMEM((CH,), jnp.int32),
        pltpu.VMEM((CH,), jnp.int32),
        pltpu.VMEM((CH,), jnp.float32),
        pltpu.VMEM((CH, HF), jnp.float32),
        pltpu.VMEM((CH,), jnp.int32),
        pltpu.VMEM((CH,), jnp.int32),
        pltpu.VMEM((CH,), jnp.float32),
        pltpu.VMEM((CH, HF), jnp.float32),
        pltpu.VMEM((CH,), jnp.int32),
        pltpu.VMEM((CH,), jnp.int32),
        pltpu.VMEM((CH,), jnp.float32),
        pltpu.VMEM((CH, HF), jnp.float32),
        pltpu.VMEM((max(tail, 16),), jnp.int32),
        pltpu.VMEM((max(tail, 16),), jnp.int32),
        pltpu.VMEM((max(tail, 16),), jnp.float32),
        pltpu.VMEM((max(tail, 16), HF), jnp.float32),
        pltpu.VMEM((CH, HF), jnp.float32),
        pltpu.VMEM_SHARED((npad, HF), jnp.float32),
        pltpu.SemaphoreType.DMA,
        pltpu.SemaphoreType.DMA,
        pltpu.SemaphoreType.DMA,
        pltpu.SemaphoreType.DMA,
        pltpu.SemaphoreType.DMA,
        pltpu.SemaphoreType.DMA,
        pltpu.SemaphoreType.DMA,
    ]

    @functools.partial(
        pl.kernel,
        out_type=[jax.ShapeDtypeStruct((npad, HF), jnp.float32)] * NC,
        mesh=_mesh(),
        scratch_types=scratch,
        compiler_params=_sc_params(),
    )
    def seg_kernel(g_hbm, src_hbm, dst_hbm, ew_hbm, o0, o1,
                   si0, di0, wi0, rv0, si1, di1, wi1, rv1,
                   si2, di2, wi2, rv2,
                   src_t, dst_t, ew_t, rows_t,
                   zb, acc, ls0, ls1, ls2, gs0, gs1, gs2, sem):
        c = lax.axis_index("c")
        s = lax.axis_index("s")
        wid = s * NC + c
        sib, dib, wib = [si0, si1, si2], [di0, di1, di2], [wi0, wi1, wi2]
        rvb = [rv0, rv1, rv2]
        lsb, gsb = [ls0, ls1, ls2], [gs0, gs1, gs2]

        def zloop(i, _):
            zb[i, :] = jnp.zeros((HF,), jnp.float32)
            return 0
        lax.fori_loop(0, CH, zloop, 0)

        row0 = pl.multiple_of(s * rpt, 8)

        def zcp(i, _):
            r = pl.multiple_of(row0 + i * CH, 8)
            pltpu.sync_copy(zb, acc.at[pl.ds(r, CH), :])
            return 0
        lax.fori_loop(0, nzr, zcp, 0)
        if ztr:
            r = pl.multiple_of(row0 + nzr * CH, 8)
            pltpu.sync_copy(zb.at[pl.ds(0, ztr), :], acc.at[pl.ds(r, ztr), :])
        plsc.subcore_barrier()

        base = wid * ept

        def issue_loads(i, b):
            o = pl.multiple_of(base + i * CH, 8)
            pltpu.async_copy(src_hbm.at[pl.ds(o, CH)], sib[b], lsb[b])
            pltpu.async_copy(dst_hbm.at[pl.ds(o, CH)], dib[b], lsb[b])
            pltpu.async_copy(ew_hbm.at[pl.ds(o, CH)], wib[b], lsb[b])

        def wait_loads(b):
            o0_ = pl.multiple_of(base, 8)
            pltpu.make_async_copy(
                src_hbm.at[pl.ds(o0_, CH)], sib[b], lsb[b]).wait()
            pltpu.make_async_copy(
                dst_hbm.at[pl.ds(o0_, CH)], dib[b], lsb[b]).wait()
            pltpu.make_async_copy(
                ew_hbm.at[pl.ds(o0_, CH)], wib[b], lsb[b]).wait()

        def issue_gather(b):
            pltpu.async_copy(g_hbm.at[sib[b]], rvb[b], gsb[b])

        def wait_gather(b):
            pltpu.make_async_copy(g_hbm.at[sib[b]], rvb[b], gsb[b]).wait()

        def scale_scatter(b):
            rv, wi, di = rvb[b], wib[b], dib[b]

            def scale(j, _):
                wv = wi[pl.ds(j * 16, 16)]
                for k in range(16):
                    w16 = wv.at[jnp.full((16,), k, jnp.int32)].get(
                        mode="promise_in_bounds")
                    r = j * 16 + k
                    rv[r, :] = rv[r, :] * w16
                return 0
            lax.fori_loop(0, CH // 16, scale, 0)
            pltpu.sync_copy(rv, acc.at[di], add=True)

        def step(i, b):
            wait_gather(b)

            @pl.when(i + 2 < nfull)
            def _():
                wait_loads((b + 2) % 3)
                issue_gather((b + 2) % 3)
            scale_scatter(b)

            @pl.when(i + 3 < nfull)
            def _():
                issue_loads(i + 3, b)

        issue_loads(0, 0)
        wait_loads(0)
        issue_gather(0)
        if nfull > 1:
            issue_loads(1, 1)
            wait_loads(1)
            issue_gather(1)
        if nfull > 2:
            issue_loads(2, 2)

        def triple(p, _):
            step(3 * p, 0)
            step(3 * p + 1, 1)
            step(3 * p + 2, 2)
            return 0
        lax.fori_loop(0, nfull // 3, triple, 0)
        for r_ in range(nfull % 3):
            i_ = nfull - (nfull % 3) + r_
            step(i_, i_ % 3)

        if tail:
            o = pl.multiple_of(base + nfull * CH, 8)
            pltpu.sync_copy(src_hbm.at[pl.ds(o, tail)], src_t)
            pltpu.sync_copy(dst_hbm.at[pl.ds(o, tail)], dst_t)
            pltpu.sync_copy(ew_hbm.at[pl.ds(o, tail)], ew_t)
            pltpu.async_copy(g_hbm.at[src_t], rows_t, sem).wait()

            def scale_t(j, _):
                wv = ew_t[pl.ds(j * 16, 16)]
                for k in range(16):
                    w16 = wv.at[jnp.full((16,), k, jnp.int32)].get(
                        mode="promise_in_bounds")
                    r = j * 16 + k
                    rows_t[r, :] = rows_t[r, :] * w16
                return 0
            lax.fori_loop(0, tail // 16, scale_t, 0)
            pltpu.sync_copy(rows_t, acc.at[dst_t], add=True)
        plsc.subcore_barrier()

        def flush(out_ref):
            def fcp(i, _):
                r = pl.multiple_of(row0 + i * CH, 8)
                pltpu.sync_copy(acc.at[pl.ds(r, CH), :], zb)
                pltpu.sync_copy(zb, out_ref.at[pl.ds(r, CH), :])
                return 0
            lax.fori_loop(0, nzr, fcp, 0)
            if ztr:
                r = pl.multiple_of(row0 + nzr * CH, 8)
                pltpu.sync_copy(acc.at[pl.ds(r, ztr), :], zb.at[pl.ds(0, ztr), :])
                pltpu.sync_copy(zb.at[pl.ds(0, ztr), :],
                                out_ref.at[pl.ds(r, ztr), :])

        @pl.when(c == 0)
        def _():
            flush(o0)

        @pl.when(c == 1)
        def _():
            flush(o1)

    return seg_kernel


@functools.cache
def _make_prep(N, DI):
    nb = N // RB

    @functools.partial(
        pl.pallas_call,
        grid=(nb,),
        in_specs=[
            pl.BlockSpec((RB, DI), lambda i: (i, 0)),
            pl.BlockSpec((RB, 1), lambda i: (i, 0)),
            pl.BlockSpec((RB, 1), lambda i: (i, 0)),
        ],
        out_specs=[
            pl.BlockSpec((RB, 1), lambda i: (i, 0)),
            pl.BlockSpec((RB, HF), lambda i: (i, 0)),
        ],
        out_shape=[
            jax.ShapeDtypeStruct((N, 1), jnp.float32),
            jax.ShapeDtypeStruct((N, HF), jnp.float32),
        ],
    )
    def prep(x_ref, d0, d1, dinv_o, y_o):
        deg = d0[...] + d1[...] + 1.0
        dinv = lax.rsqrt(deg)
        dinv_o[...] = dinv
        y_o[...] = jnp.concatenate(
            [x_ref[...] * dinv, jnp.zeros((RB, HF - DI), jnp.float32)], axis=1)

    return prep


@functools.cache
def _make_mid(N, DI, H):
    nb = N // RB
    half = pl.BlockSpec((RB, HF), lambda i: (i, 0))

    @functools.partial(
        pl.pallas_call,
        grid=(nb,),
        in_specs=[
            half, half, half,
            pl.BlockSpec((RB, 1), lambda i: (i, 0)),
            pl.BlockSpec((DI, H), lambda i: (0, 0)),
            pl.BlockSpec((1, H), lambda i: (0, 0)),
            pl.BlockSpec((H, H), lambda i: (0, 0)),
        ],
        out_specs=[half, half],
        out_shape=[
            jax.ShapeDtypeStruct((N, HF), jnp.float32),
            jax.ShapeDtypeStruct((N, HF), jnp.float32),
        ],
    )
    def mid(sa, sb, y, dinv, w1, b, w2, o0, o1):
        t = (sa[...] + sb[...] + y[...])[:, :DI]
        h = jnp.maximum(
            jnp.dot(t, w1[...], preferred_element_type=jnp.float32)
            * dinv[...] + b[...], 0.0)
        g2 = jnp.dot(h, w2[...], preferred_element_type=jnp.float32) * dinv[...]
        o0[...] = g2[:, :HF]
        o1[...] = g2[:, HF:]

    return mid


@functools.cache
def _make_fin(N, H):
    nb = N // RB
    half = pl.BlockSpec((RB, HF), lambda i: (i, 0))

    @functools.partial(
        pl.pallas_call,
        grid=(nb,),
        in_specs=[
            half, half, half, half,
            pl.BlockSpec((RB, 1), lambda i: (i, 0)),
            pl.BlockSpec((1, H), lambda i: (0, 0)),
        ],
        out_specs=pl.BlockSpec((RB, H), lambda i: (i, 0)),
        out_shape=jax.ShapeDtypeStruct((N, H), jnp.float32),
    )
    def fin(s0, s1, g0, g1, dinv, b, z_o):
        t = jnp.concatenate([s0[...] + g0[...], s1[...] + g1[...]], axis=1)
        z_o[...] = jnp.maximum(t * dinv[...] + b[...], 0.0)

    return fin


@functools.cache
def _make_head(NV, NTOT, H, CC, DEC):
    vb = NV // RB
    nb = NTOT // RB
    NK = NTOT - NV

    def cw(shape):
        return pl.BlockSpec(shape, lambda i: (0,) * len(shape))

    @functools.partial(
        pl.pallas_call,
        grid=(nb,),
        in_specs=[
            pl.BlockSpec((RB, H), lambda i: (jnp.minimum(i, vb - 1), 0)),
            pl.BlockSpec((RB, H), lambda i: (i, 0)),
            pl.BlockSpec((RB, CC), lambda i: (i, 0)),
            pl.BlockSpec((1, 1, RB), lambda i: (jnp.minimum(i, vb - 1), 0, 0)),
            cw((CC, CC)), cw((1, CC)), cw((CC, CC)), cw((1, CC)),
            cw((CC, DEC)), cw((1, DEC)), cw((1, DEC)), cw((1, 1)),
            cw((CC, DEC)), cw((1, DEC)), cw((1, DEC)), cw((1, 1)),
            cw((CC, DEC)), cw((1, DEC)), cw((1, DEC)), cw((1, 1)),
        ],
        out_specs=[
            pl.BlockSpec((RB, CC), lambda i: (i, 0)),
            pl.BlockSpec((RB, CC), lambda i: (i, 0)),
            pl.BlockSpec((RB, 1), lambda i: (jnp.minimum(i, vb - 1), 0)),
            pl.BlockSpec((RB, 1), lambda i: (jnp.maximum(i - vb, 0), 0)),
            pl.BlockSpec((RB, 1), lambda i: (jnp.minimum(i, vb - 1), 0)),
            pl.BlockSpec((NG, CC), lambda i: (0, 0)),
            pl.BlockSpec((NG, CC), lambda i: (0, 0)),
        ],
        out_shape=[
            jax.ShapeDtypeStruct((NTOT, CC), jnp.float32),
            jax.ShapeDtypeStruct((NTOT, CC), jnp.float32),
            jax.ShapeDtypeStruct((NV, 1), jnp.float32),
            jax.ShapeDtypeStruct((NK, 1), jnp.float32),
            jax.ShapeDtypeStruct((NV, 1), jnp.float32),
            jax.ShapeDtypeStruct((NG, CC), jnp.float32),
            jax.ShapeDtypeStruct((NG, CC), jnp.float32),
        ],
    )
    def head(zo, zf, eps, bv, muw, mub, lvw, lvb,
             dxw1, dxb1, dxw2, dxb2, dkw1, dkb1, dkw2, dkb2,
             diw1, dib1, diw2, dib2,
             zmu_o, zlv_o, xh_o, pk_o, ig_o, ps_o, pn_o):
        i = pl.program_id(0)
        isv = i < vb
        f = zf[...]
        c0 = jnp.where(isv, zo[...], f)
        c1 = jnp.where(isv, f, jnp.zeros_like(f))
        z = jnp.concatenate([c0, c1], axis=1)
        mu = jnp.dot(z, muw[...], preferred_element_type=jnp.float32) + mub[...]
        lv = jnp.dot(z, lvw[...], preferred_element_type=jnp.float32) + lvb[...]
        zmu_o[...] = mu
        zlv_o[...] = lv
        zz = mu + jnp.exp(0.5 * lv) * eps[...]

        @pl.when(i == 0)
        def _():
            ps_o[...] = jnp.zeros((NG, CC), jnp.float32)
            pn_o[...] = jnp.zeros((NG, CC), jnp.float32)

        @pl.when(isv)
        def _():
            h = jnp.maximum(
                jnp.dot(zz, dxw1[...], preferred_element_type=jnp.float32)
                + dxb1[...], 0.0)
            xh_o[...] = jnp.sum(h * dxw2[...], axis=1, keepdims=True) + dxb2[0, 0]
            hi = jnp.maximum(
                jnp.dot(zz, diw1[...], preferred_element_type=jnp.float32)
                + dib1[...], 0.0)
            logit = jnp.sum(hi * diw2[...], axis=1, keepdims=True) + dib2[0, 0]
            ig_o[...] = 1.0 / (1.0 + jnp.exp(-logit))
            grp = bv[0, 0, :]
            m = (lax.broadcasted_iota(jnp.int32, (NG, RB), 0)
                 == grp[None, :]).astype(jnp.float32)
            ps_o[...] += jnp.dot(m, zz, preferred_element_type=jnp.float32)
            pn_o[...] += jnp.dot(m, jnp.ones((RB, CC), jnp.float32),
                                 preferred_element_type=jnp.float32)

        @pl.when(jnp.logical_not(isv))
        def _():
            hk = jnp.maximum(
                jnp.dot(zz, dkw1[...], preferred_element_type=jnp.float32)
                + dkb1[...], 0.0)
            pk_o[...] = jnp.sum(hk * dkw2[...], axis=1, keepdims=True) + dkb2[0, 0]

    return head


@functools.cache
def _make_cost(CC, DEC):
    @functools.partial(
        pl.pallas_call,
        grid=(1,),
        in_specs=[
            pl.BlockSpec((NG, CC), lambda i: (0, 0)),
            pl.BlockSpec((NG, CC), lambda i: (0, 0)),
            pl.BlockSpec((CC, DEC), lambda i: (0, 0)),
            pl.BlockSpec((1, DEC), lambda i: (0, 0)),
            pl.BlockSpec((1, DEC), lambda i: (0, 0)),
            pl.BlockSpec((1, 1), lambda i: (0, 0)),
        ],
        out_specs=pl.BlockSpec((NG, 1), lambda i: (0, 0)),
        out_shape=jax.ShapeDtypeStruct((NG, 1), jnp.float32),
    )
    def cost(ps, pn, w1, b1, w2, b2, out):
        pooled = ps[...] / jnp.maximum(pn[...], 1.0)
        h = jnp.maximum(
            jnp.dot(pooled, w1[...], preferred_element_type=jnp.float32)
            + b1[...], 0.0)
        out[...] = jnp.sum(h * w2[...], axis=1, keepdims=True) + b2[0, 0]

    return cost


def _encode(x, src, dst, ew, W1, b1, W2, b2):
    n = x.shape[0]
    e = ew.shape[0]
    h = W1.shape[1]
    d0, d1 = _make_deg(e, n)(dst, ew)
    dinv, y = _make_prep(n, x.shape[1])(x, d0[:n, :1], d1[:n, :1])
    sa, sb = _make_segsum_split(e, n)(y, src, dst, ew)
    h0, h1 = _make_mid(n, x.shape[1], h)(
        sa[:n], sb[:n], y, dinv, W1, b1.reshape(1, -1), W2)
    t0, t1 = _make_segsum(e, n)(h0, h1, src, dst, ew)
    return _make_fin(n, h)(t0[:n], t1[:n], h0, h1, dinv, b2.reshape(1, -1))


def kernel(x_obj, edge_index_obj, edge_weight_obj, x_feas, edge_index_feas,
           edge_weight_feas, batch_var, binary_idx, eps, params):
    p = params
    n_var = batch_var.shape[0]
    n_tot = x_feas.shape[0]
    cc = eps.shape[1]
    dec = p["dx_W1"].shape[1]

    z_obj = _encode(x_obj, edge_index_obj[0], edge_index_obj[1],
                    edge_weight_obj, p["obj_W1"], p["obj_b1"],
                    p["obj_W2"], p["obj_b2"])
    zf = _encode(x_feas, edge_index_feas[0], edge_index_feas[1],
                 edge_weight_feas, p["cons_W1"], p["cons_b1"],
                 p["cons_W2"], p["cons_b2"])

    bv3 = batch_var.reshape(n_var // RB, 1, RB)
    zmu, zlv, xh, pk, ig, ps, pn = _make_head(
        n_var, n_tot, z_obj.shape[1], cc, dec)(
        z_obj, zf, eps, bv3,
        p["mu_W"], p["mu_b"].reshape(1, -1),
        p["lv_W"], p["lv_b"].reshape(1, -1),
        p["dx_W1"], p["dx_b1"].reshape(1, -1),
        p["dx_W2"].reshape(1, -1), p["dx_b2"].reshape(1, 1),
        p["dk_W1"], p["dk_b1"].reshape(1, -1),
        p["dk_W2"].reshape(1, -1), p["dk_b2"].reshape(1, 1),
        p["di_W1"], p["di_b1"].reshape(1, -1),
        p["di_W2"].reshape(1, -1), p["di_b2"].reshape(1, 1))

    cost = _make_cost(cc, dec)(
        ps, pn, p["dc_W1"], p["dc_b1"].reshape(1, -1),
        p["dc_W2"].reshape(1, -1), p["dc_b2"].reshape(1, 1))
    pint = _make_take(binary_idx.shape[0])(ig.reshape(-1), binary_idx)
    return (xh.reshape(-1), cost.reshape(-1), pk.reshape(-1), pint, zmu, zlv)
